# Initial kernel scaffold; baseline (speedup 1.0000x reference)
import functools

import jax
import jax.numpy as jnp
from jax import lax
from jax.experimental import pallas as pl
from jax.experimental.pallas import tpu as pltpu
from jax.experimental.pallas import tpu_sc as plsc

NC = 2
NS = 16
LN = 16
K = 128
H = 64


def _mesh():
    return plsc.VectorSubcoreMesh(core_axis_name="c", subcore_axis_name="s")


def _f32(*shape):
    return jax.ShapeDtypeStruct(shape, jnp.float32)


def _rows_op(body, k):
    def outer(r, _):
        for j in range(H // LN):
            body(r, pl.ds(j * LN, LN))
        return _
    lax.fori_loop(0, k, outer, None)


def _chunk_loop(s, n_edges, body):
    total = n_edges // K
    n = (total // NS) + jnp.where(s < (total % NS), 1, 0)

    def step(i, _):
        body((s + i * NS) * K)
        return _
    lax.fori_loop(0, n, step, None)


def _zero_acc(s, acc, zbuf, n_rows):
    def z(r, _):
        for j in range(H // LN):
            zbuf[r, pl.ds(j * LN, LN)] = jnp.zeros((LN,), jnp.float32)
        return _
    lax.fori_loop(0, K, z, None)
    per = n_rows // NS
    r0 = s * per
    off = 0
    while off < per:
        kk = min(K, per - off)
        pltpu.sync_copy(zbuf.at[pl.ds(0, kk)], acc.at[pl.ds(r0 + off, kk)])
        off += kk


def _sc_init(node_feats, edge_feats, edge_index):
    n, d = node_feats.shape
    e = edge_feats.shape[0]

    @functools.partial(
        pl.kernel,
        out_type=(_f32(e, d), _f32(e, H), _f32(e, H)),
        mesh=_mesh(),
        scratch_types=[
            pltpu.VMEM_SHARED((n, H), jnp.float32),
            pltpu.VMEM((K,), jnp.int32),
            pltpu.VMEM((K, H), jnp.float32),
            pltpu.VMEM((K, H), jnp.float32),
            pltpu.SemaphoreType.DMA,
            pltpu.SemaphoreType.DMA,
        ],
    )
    def body(nf_hbm, ef_hbm, ei_hbm, eh_hbm, h0_hbm, h1_hbm,
             nf_sh, idxb, nfb, efb, sem_a, sem_b):
        c = lax.axis_index("c")
        s = lax.axis_index("s")
        per = n // NS
        r0 = s * per
        pltpu.sync_copy(nf_hbm.at[pl.ds(r0, per), pl.ds(c * H, H)],
                        nf_sh.at[pl.ds(r0, per)])
        plsc.subcore_barrier()

        def chunk(e0):
            pltpu.sync_copy(ei_hbm.at[0, pl.ds(e0, K)], idxb)
            cp1 = pltpu.async_copy(nf_sh.at[idxb], nfb, sem_a)
            cp2 = pltpu.async_copy(ef_hbm.at[pl.ds(e0, K), pl.ds(c * H, H)],
                                   efb, sem_b)
            cp1.wait()
            cp2.wait()

            def op(r, sl):
                eh = nfb[r, sl] + efb[r, sl]
                efb[r, sl] = eh
                nfb[r, sl] = jnp.maximum(eh, 0.0)
            _rows_op(op, K)
            pltpu.sync_copy(efb, eh_hbm.at[pl.ds(e0, K), pl.ds(c * H, H)])

            @pl.when(c == 0)
            def _():
                pltpu.sync_copy(nfb, h0_hbm.at[pl.ds(e0, K)])

            @pl.when(c == 1)
            def _():
                pltpu.sync_copy(nfb, h1_hbm.at[pl.ds(e0, K)])

        _chunk_loop(s, e, chunk)

    return body(node_feats, edge_feats, edge_index)


def _sc_message(h0, h1, edge_index, rev_index, n_nodes):
    e = h0.shape[0]

    @functools.partial(
        pl.kernel,
        out_type=(_f32(e, H), _f32(e, H)),
        mesh=_mesh(),
        scratch_types=[
            pltpu.VMEM_SHARED((n_nodes, H), jnp.float32),
            pltpu.VMEM((K,), jnp.int32),
            pltpu.VMEM((K,), jnp.int32),
            pltpu.VMEM((K,), jnp.int32),
            pltpu.VMEM((K, H), jnp.float32),
            pltpu.VMEM((K, H), jnp.float32),
            pltpu.VMEM((K, H), jnp.float32),
            pltpu.SemaphoreType.DMA,
            pltpu.SemaphoreType.DMA,
        ],
    )
    def body(h0_hbm, h1_hbm, ei_hbm, rev_hbm, em0_hbm, em1_hbm,
             acc, didx, sidx, ridx, rowsb, nmb, hrb, sem_a, sem_b):
        c = lax.axis_index("c")
        s = lax.axis_index("s")

        def run(h_hbm, em_hbm):
            _zero_acc(s, acc, rowsb, n_nodes)
            plsc.subcore_barrier()

            def scatter_chunk(e0):
                pltpu.sync_copy(ei_hbm.at[1, pl.ds(e0, K)], didx)
                pltpu.sync_copy(h_hbm.at[pl.ds(e0, K)], rowsb)
                pltpu.sync_copy(rowsb, acc.at[didx], add=True)

            _chunk_loop(s, e, scatter_chunk)
            plsc.subcore_barrier()

            def gather_chunk(e0):
                pltpu.sync_copy(ei_hbm.at[0, pl.ds(e0, K)], sidx)
                pltpu.sync_copy(rev_hbm.at[pl.ds(e0, K)], ridx)
                cp1 = pltpu.async_copy(acc.at[sidx], nmb, sem_a)
                cp2 = pltpu.async_copy(h_hbm.at[ridx], hrb, sem_b)
                cp1.wait()
                cp2.wait()

                def op(r, sl):
                    nmb[r, sl] = nmb[r, sl] - hrb[r, sl]
                _rows_op(op, K)
                pltpu.sync_copy(nmb, em_hbm.at[pl.ds(e0, K)])

            _chunk_loop(s, e, gather_chunk)

        @pl.when(c == 0)
        def _():
            run(h0_hbm, em0_hbm)

        @pl.when(c == 1)
        def _():
            run(h1_hbm, em1_hbm)

    return body(h0, h1, edge_index, rev_index)


def _sc_final(eh, edge_index, n_nodes):
    e, d = eh.shape

    @functools.partial(
        pl.kernel,
        out_type=_f32(n_nodes, d),
        mesh=_mesh(),
        scratch_types=[
            pltpu.VMEM_SHARED((n_nodes, H), jnp.float32),
            pltpu.VMEM((K,), jnp.int32),
            pltpu.VMEM((K, H), jnp.float32),
        ],
    )
    def body(eh_hbm, ei_hbm, nh_hbm, acc, didx, rowsb):
        c = lax.axis_index("c")
        s = lax.axis_index("s")
        _zero_acc(s, acc, rowsb, n_nodes)
        plsc.subcore_barrier()

        def chunk(e0):
            pltpu.sync_copy(ei_hbm.at[1, pl.ds(e0, K)], didx)
            pltpu.sync_copy(eh_hbm.at[pl.ds(e0, K), pl.ds(c * H, H)], rowsb)
            pltpu.sync_copy(rowsb, acc.at[didx], add=True)

        _chunk_loop(s, e, chunk)
        plsc.subcore_barrier()
        per = n_nodes // NS
        r0 = s * per
        pltpu.sync_copy(acc.at[pl.ds(r0, per)],
                        nh_hbm.at[pl.ds(r0, per), pl.ds(c * H, H)])

    return body(eh, edge_index)


def _tc_layer(eh, em0, em1, W, b, last):
    e, d = eh.shape
    be = 3200
    grid = (e // be,)
    b2 = b.reshape(1, d)

    def mm_body(eh_ref, em0_ref, em1_ref, w_ref, b_ref, *outs):
        w = w_ref[...]
        upd = jnp.dot(em0_ref[...], w[:H, :], preferred_element_type=jnp.float32)
        upd = upd + jnp.dot(em1_ref[...], w[H:, :],
                            preferred_element_type=jnp.float32)
        ehn = eh_ref[...] + upd + b_ref[...]
        outs[0][...] = ehn
        if not last:
            h = jnp.maximum(ehn, 0.0)
            outs[1][...] = h[:, :H]
            outs[2][...] = h[:, H:]

    out_shape = [_f32(e, d)]
    out_specs = [pl.BlockSpec((be, d), lambda i: (i, 0))]
    if not last:
        out_shape += [_f32(e, H), _f32(e, H)]
        out_specs += [pl.BlockSpec((be, H), lambda i: (i, 0))] * 2

    res = pl.pallas_call(
        mm_body,
        grid=grid,
        in_specs=[
            pl.BlockSpec((be, d), lambda i: (i, 0)),
            pl.BlockSpec((be, H), lambda i: (i, 0)),
            pl.BlockSpec((be, H), lambda i: (i, 0)),
            pl.BlockSpec((d, d), lambda i: (0, 0)),
            pl.BlockSpec((1, d), lambda i: (0, 0)),
        ],
        out_specs=out_specs,
        out_shape=out_shape,
    )(eh, em0, em1, W, b2)
    return res if not last else (res,)


def kernel(node_feats, edge_feats, edge_index, rev_index,
           W0, b0, W1, b1, W2, b2):
    n_nodes = node_feats.shape[0]
    eh, h0, h1 = _sc_init(node_feats, edge_feats, edge_index)
    for i, (W, b) in enumerate(((W0, b0), (W1, b1), (W2, b2))):
        em0, em1 = _sc_message(h0, h1, edge_index, rev_index, n_nodes)
        if i < 2:
            eh, h0, h1 = _tc_layer(eh, em0, em1, W, b, last=False)
        else:
            (eh,) = _tc_layer(eh, em0, em1, W, b, last=True)
    node_hiddens = _sc_final(eh, edge_index, n_nodes)
    return (node_hiddens, eh)


# speedup vs baseline: 1.3793x; 1.3793x over previous
import functools

import jax
import jax.numpy as jnp
from jax import lax
from jax.experimental import pallas as pl
from jax.experimental.pallas import tpu as pltpu
from jax.experimental.pallas import tpu_sc as plsc

NC = 2
NS = 16
LN = 16
K = 128
H = 64


def _mesh():
    return plsc.VectorSubcoreMesh(core_axis_name="c", subcore_axis_name="s")


_SC_PARAMS = pltpu.CompilerParams(use_tc_tiling_on_sc=False)


def _f32(*shape):
    return jax.ShapeDtypeStruct(shape, jnp.float32)


def _rows_op(body, k):
    def outer(r, _):
        for j in range(H // LN):
            body(r, pl.ds(j * LN, LN))
        return _
    lax.fori_loop(0, k, outer, None)


def _chunk_loop(s, n_edges, body):
    total = n_edges // K
    n = (total // NS) + jnp.where(s < (total % NS), 1, 0)

    def step(i, _):
        body((s + i * NS) * K)
        return _
    lax.fori_loop(0, n, step, None)


def _zero_acc(s, acc, zbuf, n_rows):
    def z(r, _):
        for j in range(H // LN):
            zbuf[r, pl.ds(j * LN, LN)] = jnp.zeros((LN,), jnp.float32)
        return _
    lax.fori_loop(0, K, z, None)
    per = n_rows // NS
    r0 = s * per
    off = 0
    while off < per:
        kk = min(K, per - off)
        pltpu.sync_copy(zbuf.at[pl.ds(0, kk)], acc.at[pl.ds(r0 + off, kk)])
        off += kk


def _sc_init(node_feats, edge_feats, edge_index):
    n, d = node_feats.shape
    e = edge_feats.shape[0]

    @functools.partial(
        pl.kernel,
        out_type=(_f32(e, d), _f32(e, H), _f32(e, H)),
        mesh=_mesh(),
        compiler_params=_SC_PARAMS,
        scratch_types=[
            pltpu.VMEM_SHARED((n, H), jnp.float32),
            pltpu.VMEM((K,), jnp.int32),
            pltpu.VMEM((K, H), jnp.float32),
            pltpu.VMEM((K, H), jnp.float32),
            pltpu.SemaphoreType.DMA,
            pltpu.SemaphoreType.DMA,
        ],
    )
    def body(nf_hbm, ef_hbm, ei_hbm, eh_hbm, h0_hbm, h1_hbm,
             nf_sh, idxb, nfb, efb, sem_a, sem_b):
        c = lax.axis_index("c")
        s = lax.axis_index("s")
        per = n // NS
        r0 = s * per
        pltpu.sync_copy(nf_hbm.at[pl.ds(r0, per), pl.ds(c * H, H)],
                        nf_sh.at[pl.ds(r0, per)])
        plsc.subcore_barrier()

        def chunk(e0):
            pltpu.sync_copy(ei_hbm.at[0, pl.ds(e0, K)], idxb)
            cp1 = pltpu.async_copy(nf_sh.at[idxb], nfb, sem_a)
            cp2 = pltpu.async_copy(ef_hbm.at[pl.ds(e0, K), pl.ds(c * H, H)],
                                   efb, sem_b)
            cp1.wait()
            cp2.wait()

            def op(r, sl):
                eh = nfb[r, sl] + efb[r, sl]
                efb[r, sl] = eh
                nfb[r, sl] = jnp.maximum(eh, 0.0)
            _rows_op(op, K)
            pltpu.sync_copy(efb, eh_hbm.at[pl.ds(e0, K), pl.ds(c * H, H)])

            @pl.when(c == 0)
            def _():
                pltpu.sync_copy(nfb, h0_hbm.at[pl.ds(e0, K)])

            @pl.when(c == 1)
            def _():
                pltpu.sync_copy(nfb, h1_hbm.at[pl.ds(e0, K)])

        _chunk_loop(s, e, chunk)

    return body(node_feats, edge_feats, edge_index)


def _sc_message(h0, h1, edge_index, rev_index, n_nodes):
    e = h0.shape[0]

    @functools.partial(
        pl.kernel,
        out_type=(_f32(e, H), _f32(e, H)),
        mesh=_mesh(),
        compiler_params=_SC_PARAMS,
        scratch_types=[
            pltpu.VMEM_SHARED((n_nodes, H), jnp.float32),
            pltpu.VMEM((K,), jnp.int32),
            pltpu.VMEM((K,), jnp.int32),
            pltpu.VMEM((K,), jnp.int32),
            pltpu.VMEM((K, H), jnp.float32),
            pltpu.VMEM((K, H), jnp.float32),
            pltpu.VMEM((K, H), jnp.float32),
            pltpu.SemaphoreType.DMA,
            pltpu.SemaphoreType.DMA,
        ],
    )
    def body(h0_hbm, h1_hbm, ei_hbm, rev_hbm, em0_hbm, em1_hbm,
             acc, didx, sidx, ridx, rowsb, nmb, hrb, sem_a, sem_b):
        c = lax.axis_index("c")
        s = lax.axis_index("s")

        def run(h_hbm, em_hbm):
            _zero_acc(s, acc, rowsb, n_nodes)
            plsc.subcore_barrier()

            def scatter_chunk(e0):
                pltpu.sync_copy(ei_hbm.at[1, pl.ds(e0, K)], didx)
                pltpu.sync_copy(h_hbm.at[pl.ds(e0, K)], rowsb)
                pltpu.sync_copy(rowsb, acc.at[didx], add=True)

            _chunk_loop(s, e, scatter_chunk)
            plsc.subcore_barrier()

            def gather_chunk(e0):
                pltpu.sync_copy(ei_hbm.at[0, pl.ds(e0, K)], sidx)
                pltpu.sync_copy(rev_hbm.at[pl.ds(e0, K)], ridx)
                cp1 = pltpu.async_copy(acc.at[sidx], nmb, sem_a)
                cp2 = pltpu.async_copy(h_hbm.at[ridx], hrb, sem_b)
                cp1.wait()
                cp2.wait()

                def op(r, sl):
                    nmb[r, sl] = nmb[r, sl] - hrb[r, sl]
                _rows_op(op, K)
                pltpu.sync_copy(nmb, em_hbm.at[pl.ds(e0, K)])

            _chunk_loop(s, e, gather_chunk)

        @pl.when(c == 0)
        def _():
            run(h0_hbm, em0_hbm)

        @pl.when(c == 1)
        def _():
            run(h1_hbm, em1_hbm)

    return body(h0, h1, edge_index, rev_index)


def _sc_final(eh, edge_index, n_nodes):
    e, d = eh.shape

    @functools.partial(
        pl.kernel,
        out_type=_f32(n_nodes, d),
        mesh=_mesh(),
        compiler_params=_SC_PARAMS,
        scratch_types=[
            pltpu.VMEM_SHARED((n_nodes, H), jnp.float32),
            pltpu.VMEM((K,), jnp.int32),
            pltpu.VMEM((K, H), jnp.float32),
        ],
    )
    def body(eh_hbm, ei_hbm, nh_hbm, acc, didx, rowsb):
        c = lax.axis_index("c")
        s = lax.axis_index("s")
        _zero_acc(s, acc, rowsb, n_nodes)
        plsc.subcore_barrier()

        def chunk(e0):
            pltpu.sync_copy(ei_hbm.at[1, pl.ds(e0, K)], didx)
            pltpu.sync_copy(eh_hbm.at[pl.ds(e0, K), pl.ds(c * H, H)], rowsb)
            pltpu.sync_copy(rowsb, acc.at[didx], add=True)

        _chunk_loop(s, e, chunk)
        plsc.subcore_barrier()
        per = n_nodes // NS
        r0 = s * per
        pltpu.sync_copy(acc.at[pl.ds(r0, per)],
                        nh_hbm.at[pl.ds(r0, per), pl.ds(c * H, H)])

    return body(eh, edge_index)


def _tc_layer(eh, em0, em1, W, b, last):
    e, d = eh.shape
    be = 3200
    grid = (e // be,)
    b2 = b.reshape(1, d)

    def mm_body(eh_ref, em0_ref, em1_ref, w_ref, b_ref, *outs):
        w = w_ref[...]
        upd = jnp.dot(em0_ref[...], w[:H, :], preferred_element_type=jnp.float32)
        upd = upd + jnp.dot(em1_ref[...], w[H:, :],
                            preferred_element_type=jnp.float32)
        ehn = eh_ref[...] + upd + b_ref[...]
        outs[0][...] = ehn
        if not last:
            h = jnp.maximum(ehn, 0.0)
            outs[1][...] = h[:, :H]
            outs[2][...] = h[:, H:]

    out_shape = [_f32(e, d)]
    out_specs = [pl.BlockSpec((be, d), lambda i: (i, 0))]
    if not last:
        out_shape += [_f32(e, H), _f32(e, H)]
        out_specs += [pl.BlockSpec((be, H), lambda i: (i, 0))] * 2

    res = pl.pallas_call(
        mm_body,
        grid=grid,
        in_specs=[
            pl.BlockSpec((be, d), lambda i: (i, 0)),
            pl.BlockSpec((be, H), lambda i: (i, 0)),
            pl.BlockSpec((be, H), lambda i: (i, 0)),
            pl.BlockSpec((d, d), lambda i: (0, 0)),
            pl.BlockSpec((1, d), lambda i: (0, 0)),
        ],
        out_specs=out_specs,
        out_shape=out_shape,
    )(eh, em0, em1, W, b2)
    return tuple(res)


def kernel(node_feats, edge_feats, edge_index, rev_index,
           W0, b0, W1, b1, W2, b2):
    n_nodes = node_feats.shape[0]
    eh, h0, h1 = _sc_init(node_feats, edge_feats, edge_index)
    for i, (W, b) in enumerate(((W0, b0), (W1, b1), (W2, b2))):
        em0, em1 = _sc_message(h0, h1, edge_index, rev_index, n_nodes)
        if i < 2:
            eh, h0, h1 = _tc_layer(eh, em0, em1, W, b, last=False)
        else:
            (eh,) = _tc_layer(eh, em0, em1, W, b, last=True)
    node_hiddens = _sc_final(eh, edge_index, n_nodes)
    return (node_hiddens, eh)


# speedup vs baseline: 1.9182x; 1.3907x over previous
import functools

import jax
import jax.numpy as jnp
from jax import lax
from jax.experimental import pallas as pl
from jax.experimental.pallas import tpu as pltpu
from jax.experimental.pallas import tpu_sc as plsc

NC = 2
NS = 16
LN = 16
K = 128
H = 64


def _mesh():
    return plsc.VectorSubcoreMesh(core_axis_name="c", subcore_axis_name="s")


_SC_PARAMS = pltpu.CompilerParams(use_tc_tiling_on_sc=False)


def _f32(*shape):
    return jax.ShapeDtypeStruct(shape, jnp.float32)


def _rows_op(body, k):
    def outer(r, _):
        for j in range(H // LN):
            body(r, pl.ds(j * LN, LN))
        return _
    lax.fori_loop(0, k, outer, None)


def _zero_acc(s, acc, zbuf, n_rows):
    def z(r, _):
        for j in range(H // LN):
            zbuf[r, pl.ds(j * LN, LN)] = jnp.zeros((LN,), jnp.float32)
        return _
    lax.fori_loop(0, K, z, None)
    per = n_rows // NS
    r0 = s * per
    off = 0
    while off < per:
        kk = min(K, per - off)
        pltpu.sync_copy(zbuf.at[pl.ds(0, kk)], acc.at[pl.ds(r0 + off, kk)])
        off += kk


def _sc_init(node_feats, edge_feats, edge_index):
    n, d = node_feats.shape
    e = edge_feats.shape[0]
    nchunk = e // K
    nmain = nchunk // NS
    nleft = nchunk - nmain * NS
    assert nmain % 2 == 0 and nmain >= 6 and e % K == 0 and n % NS == 0

    @functools.partial(
        pl.kernel,
        out_type=(_f32(e, d), _f32(e, H), _f32(e, H)),
        mesh=_mesh(),
        compiler_params=_SC_PARAMS,
        scratch_types=[
            pltpu.VMEM_SHARED((n, H), jnp.float32),
            pltpu.VMEM((K,), jnp.int32),
            pltpu.VMEM((K,), jnp.int32),
            pltpu.VMEM((K, H), jnp.float32),
            pltpu.VMEM((K, H), jnp.float32),
            pltpu.VMEM((K, H), jnp.float32),
            pltpu.VMEM((K, H), jnp.float32),
            pltpu.SemaphoreType.DMA,
            pltpu.SemaphoreType.DMA,
            pltpu.SemaphoreType.DMA,
            pltpu.SemaphoreType.DMA,
            pltpu.SemaphoreType.DMA,
            pltpu.SemaphoreType.DMA,
            pltpu.SemaphoreType.DMA,
            pltpu.SemaphoreType.DMA,
        ],
    )
    def body(nf_hbm, ef_hbm, ei_hbm, eh_hbm, h0_hbm, h1_hbm,
             nf_sh, sidx0, sidx1, nfb0, nfb1, efb0, efb1,
             isem0, isem1, esem0, esem1, oe0, oe1, oh0, oh1):
        c = lax.axis_index("c")
        s = lax.axis_index("s")
        sidx = (sidx0, sidx1)
        nfb = (nfb0, nfb1)
        efb = (efb0, efb1)
        isem = (isem0, isem1)
        esem = (esem0, esem1)
        oe = (oe0, oe1)
        oh = (oh0, oh1)

        per = n // NS
        r0 = s * per
        pltpu.sync_copy(nf_hbm.at[pl.ds(r0, per), pl.ds(c * H, H)],
                        nf_sh.at[pl.ds(r0, per)])
        plsc.subcore_barrier()

        def e0_of(i):
            return (s + i * NS) * K

        def start_in(i, b):
            pltpu.async_copy(ei_hbm.at[0, pl.ds(e0_of(i), K)],
                             sidx[b], isem[b])
            pltpu.async_copy(ef_hbm.at[pl.ds(e0_of(i), K), pl.ds(c * H, H)],
                             efb[b], esem[b])

        def wait_in(i, b):
            pltpu.make_async_copy(ei_hbm.at[0, pl.ds(e0_of(i), K)],
                                  sidx[b], isem[b]).wait()
            pltpu.make_async_copy(
                ef_hbm.at[pl.ds(e0_of(i), K), pl.ds(c * H, H)],
                efb[b], esem[b]).wait()

        def wait_out(i, b):
            pltpu.make_async_copy(
                efb[b], eh_hbm.at[pl.ds(e0_of(i), K), pl.ds(c * H, H)],
                oe[b]).wait()

            @pl.when(c == 0)
            def _():
                pltpu.make_async_copy(
                    nfb[b], h0_hbm.at[pl.ds(e0_of(i), K)], oh[b]).wait()

            @pl.when(c == 1)
            def _():
                pltpu.make_async_copy(
                    nfb[b], h1_hbm.at[pl.ds(e0_of(i), K)], oh[b]).wait()

        def step(i, b, wait_o, issue_next):
            if issue_next:
                if wait_o:
                    wait_out(i - 1, 1 - b)
                start_in(i + 1, 1 - b)
            wait_in(i, b)
            pltpu.async_copy(nf_sh.at[sidx[b]], nfb[b], isem[b]).wait()

            def op(r, sl):
                eh = nfb[b][r, sl] + efb[b][r, sl]
                efb[b][r, sl] = eh
                nfb[b][r, sl] = jnp.maximum(eh, 0.0)
            _rows_op(op, K)

            pltpu.async_copy(efb[b],
                             eh_hbm.at[pl.ds(e0_of(i), K), pl.ds(c * H, H)],
                             oe[b])

            @pl.when(c == 0)
            def _():
                pltpu.async_copy(nfb[b], h0_hbm.at[pl.ds(e0_of(i), K)],
                                 oh[b])

            @pl.when(c == 1)
            def _():
                pltpu.async_copy(nfb[b], h1_hbm.at[pl.ds(e0_of(i), K)],
                                 oh[b])

        start_in(0, 0)
        step(0, 0, False, True)
        step(1, 1, True, True)

        def grp(g, _):
            step(2 * g, 0, True, True)
            step(2 * g + 1, 1, True, True)
            return _
        lax.fori_loop(1, nmain // 2 - 1, grp, None)
        step(nmain - 2, 0, True, True)
        step(nmain - 1, 1, False, False)
        wait_out(nmain - 2, 0)
        wait_out(nmain - 1, 1)

        if nleft:
            @pl.when(s < nleft)
            def _():
                e0 = (nmain * NS + s) * K
                pltpu.sync_copy(ei_hbm.at[0, pl.ds(e0, K)], sidx0)
                cp1 = pltpu.async_copy(nf_sh.at[sidx0], nfb0, isem0)
                cp2 = pltpu.async_copy(
                    ef_hbm.at[pl.ds(e0, K), pl.ds(c * H, H)], efb0, esem0)
                cp1.wait()
                cp2.wait()

                def op(r, sl):
                    eh = nfb0[r, sl] + efb0[r, sl]
                    efb0[r, sl] = eh
                    nfb0[r, sl] = jnp.maximum(eh, 0.0)
                _rows_op(op, K)
                pltpu.sync_copy(
                    efb0, eh_hbm.at[pl.ds(e0, K), pl.ds(c * H, H)])

                @pl.when(c == 0)
                def _():
                    pltpu.sync_copy(nfb0, h0_hbm.at[pl.ds(e0, K)])

                @pl.when(c == 1)
                def _():
                    pltpu.sync_copy(nfb0, h1_hbm.at[pl.ds(e0, K)])

    return body(node_feats, edge_feats, edge_index)


def _sc_message(h0, h1, edge_index, rev_index, n_nodes):
    e = h0.shape[0]
    nchunk = e // K
    nmain = nchunk // NS
    nleft = nchunk - nmain * NS
    assert nmain % 2 == 0 and nmain >= 6

    @functools.partial(
        pl.kernel,
        out_type=(_f32(e, H), _f32(e, H)),
        mesh=_mesh(),
        compiler_params=_SC_PARAMS,
        scratch_types=[
            pltpu.VMEM_SHARED((n_nodes, H), jnp.float32),
            pltpu.VMEM((K,), jnp.int32),
            pltpu.VMEM((K,), jnp.int32),
            pltpu.VMEM((K,), jnp.int32),
            pltpu.VMEM((K,), jnp.int32),
            pltpu.VMEM((K, H), jnp.float32),
            pltpu.VMEM((K, H), jnp.float32),
            pltpu.VMEM((K, H), jnp.float32),
            pltpu.VMEM((K, H), jnp.float32),
            pltpu.SemaphoreType.DMA,
            pltpu.SemaphoreType.DMA,
            pltpu.SemaphoreType.DMA,
            pltpu.SemaphoreType.DMA,
            pltpu.SemaphoreType.DMA,
            pltpu.SemaphoreType.DMA,
            pltpu.SemaphoreType.DMA,
            pltpu.SemaphoreType.DMA,
        ],
    )
    def body(h0_hbm, h1_hbm, ei_hbm, rev_hbm, em0_hbm, em1_hbm,
             acc, aidx0, aidx1, ridx0, ridx1, rowa0, rowa1, rowb0, rowb1,
             isem0, isem1, rsem0, rsem1, hsem0, hsem1, osem0, osem1):
        c = lax.axis_index("c")
        s = lax.axis_index("s")
        aidx = (aidx0, aidx1)
        ridx = (ridx0, ridx1)
        rowa = (rowa0, rowa1)
        rowb = (rowb0, rowb1)
        isem = (isem0, isem1)
        rsem = (rsem0, rsem1)
        hsem = (hsem0, hsem1)
        osem = (osem0, osem1)

        def e0_of(i):
            return (s + i * NS) * K

        def run(h_hbm, em_hbm):
            _zero_acc(s, acc, rowa0, n_nodes)
            plsc.subcore_barrier()

            def start_arrival(i, b):
                pltpu.async_copy(ei_hbm.at[1, pl.ds(e0_of(i), K)],
                                 aidx[b], isem[b])
                pltpu.async_copy(h_hbm.at[pl.ds(e0_of(i), K)],
                                 rowa[b], hsem[b])

            def scatter_step(i, b, issue_next):
                if issue_next:
                    start_arrival(i + 1, 1 - b)
                pltpu.make_async_copy(ei_hbm.at[1, pl.ds(e0_of(i), K)],
                                      aidx[b], isem[b]).wait()
                pltpu.make_async_copy(h_hbm.at[pl.ds(e0_of(i), K)],
                                      rowa[b], hsem[b]).wait()
                pltpu.sync_copy(rowa[b], acc.at[aidx[b]], add=True)

            start_arrival(0, 0)

            def sgrp(g, _):
                scatter_step(2 * g, 0, True)
                scatter_step(2 * g + 1, 1, True)
                return _
            lax.fori_loop(0, nmain // 2 - 1, sgrp, None)
            scatter_step(nmain - 2, 0, True)
            scatter_step(nmain - 1, 1, False)

            if nleft:
                @pl.when(s < nleft)
                def _():
                    e0 = (nmain * NS + s) * K
                    pltpu.sync_copy(ei_hbm.at[1, pl.ds(e0, K)], aidx0)
                    pltpu.sync_copy(h_hbm.at[pl.ds(e0, K)], rowa0)
                    pltpu.sync_copy(rowa0, acc.at[aidx0], add=True)

            plsc.subcore_barrier()

            def start_idx(i, b):
                pltpu.async_copy(ei_hbm.at[0, pl.ds(e0_of(i), K)],
                                 aidx[b], isem[b])
                pltpu.async_copy(rev_hbm.at[pl.ds(e0_of(i), K)],
                                 ridx[b], rsem[b])

            def wait_idx(i, b):
                pltpu.make_async_copy(ei_hbm.at[0, pl.ds(e0_of(i), K)],
                                      aidx[b], isem[b]).wait()
                pltpu.make_async_copy(rev_hbm.at[pl.ds(e0_of(i), K)],
                                      ridx[b], rsem[b]).wait()

            def wait_out(i, b):
                pltpu.make_async_copy(
                    rowa[b], em_hbm.at[pl.ds(e0_of(i), K)], osem[b]).wait()

            def gather_step(i, b, wait_o, issue_idx):
                if wait_o:
                    wait_out(i - 2, b)
                wait_idx(i, b)
                cp1 = pltpu.async_copy(acc.at[aidx[b]], rowa[b], hsem[b])
                cp2 = pltpu.async_copy(h_hbm.at[ridx[b]], rowb[b], hsem[1 - b])
                cp1.wait()
                cp2.wait()
                if issue_idx:
                    start_idx(i + 2, b)

                def op(r, sl):
                    rowa[b][r, sl] = rowa[b][r, sl] - rowb[b][r, sl]
                _rows_op(op, K)
                pltpu.async_copy(rowa[b], em_hbm.at[pl.ds(e0_of(i), K)],
                                 osem[b])

            start_idx(0, 0)
            start_idx(1, 1)
            gather_step(0, 0, False, True)
            gather_step(1, 1, False, True)

            def ggrp(g, _):
                gather_step(2 * g, 0, True, True)
                gather_step(2 * g + 1, 1, True, True)
                return _
            lax.fori_loop(1, nmain // 2 - 1, ggrp, None)
            gather_step(nmain - 2, 0, True, False)
            gather_step(nmain - 1, 1, True, False)
            wait_out(nmain - 2, 0)
            wait_out(nmain - 1, 1)

            if nleft:
                @pl.when(s < nleft)
                def _():
                    e0 = (nmain * NS + s) * K
                    pltpu.sync_copy(ei_hbm.at[0, pl.ds(e0, K)], aidx0)
                    pltpu.sync_copy(rev_hbm.at[pl.ds(e0, K)], ridx0)
                    cp1 = pltpu.async_copy(acc.at[aidx0], rowa0, hsem0)
                    cp2 = pltpu.async_copy(h_hbm.at[ridx0], rowb0, hsem1)
                    cp1.wait()
                    cp2.wait()

                    def op(r, sl):
                        rowa0[r, sl] = rowa0[r, sl] - rowb0[r, sl]
                    _rows_op(op, K)
                    pltpu.sync_copy(rowa0, em_hbm.at[pl.ds(e0, K)])

        @pl.when(c == 0)
        def _():
            run(h0_hbm, em0_hbm)

        @pl.when(c == 1)
        def _():
            run(h1_hbm, em1_hbm)

    return body(h0, h1, edge_index, rev_index)


def _sc_final(eh, edge_index, n_nodes):
    e, d = eh.shape
    nchunk = e // K
    nmain = nchunk // NS
    nleft = nchunk - nmain * NS
    assert nmain % 2 == 0 and nmain >= 6

    @functools.partial(
        pl.kernel,
        out_type=_f32(n_nodes, d),
        mesh=_mesh(),
        compiler_params=_SC_PARAMS,
        scratch_types=[
            pltpu.VMEM_SHARED((n_nodes, H), jnp.float32),
            pltpu.VMEM((K,), jnp.int32),
            pltpu.VMEM((K,), jnp.int32),
            pltpu.VMEM((K, H), jnp.float32),
            pltpu.VMEM((K, H), jnp.float32),
            pltpu.SemaphoreType.DMA,
            pltpu.SemaphoreType.DMA,
            pltpu.SemaphoreType.DMA,
            pltpu.SemaphoreType.DMA,
        ],
    )
    def body(eh_hbm, ei_hbm, nh_hbm,
             acc, aidx0, aidx1, rowa0, rowa1, isem0, isem1, hsem0, hsem1):
        c = lax.axis_index("c")
        s = lax.axis_index("s")
        aidx = (aidx0, aidx1)
        rowa = (rowa0, rowa1)
        isem = (isem0, isem1)
        hsem = (hsem0, hsem1)
        _zero_acc(s, acc, rowa0, n_nodes)
        plsc.subcore_barrier()

        def e0_of(i):
            return (s + i * NS) * K

        def start_arrival(i, b):
            pltpu.async_copy(ei_hbm.at[1, pl.ds(e0_of(i), K)],
                             aidx[b], isem[b])
            pltpu.async_copy(eh_hbm.at[pl.ds(e0_of(i), K), pl.ds(c * H, H)],
                             rowa[b], hsem[b])

        def scatter_step(i, b, issue_next):
            if issue_next:
                start_arrival(i + 1, 1 - b)
            pltpu.make_async_copy(ei_hbm.at[1, pl.ds(e0_of(i), K)],
                                  aidx[b], isem[b]).wait()
            pltpu.make_async_copy(
                eh_hbm.at[pl.ds(e0_of(i), K), pl.ds(c * H, H)],
                rowa[b], hsem[b]).wait()
            pltpu.sync_copy(rowa[b], acc.at[aidx[b]], add=True)

        start_arrival(0, 0)

        def sgrp(g, _):
            scatter_step(2 * g, 0, True)
            scatter_step(2 * g + 1, 1, True)
            return _
        lax.fori_loop(0, nmain // 2 - 1, sgrp, None)
        scatter_step(nmain - 2, 0, True)
        scatter_step(nmain - 1, 1, False)

        if nleft:
            @pl.when(s < nleft)
            def _():
                e0 = (nmain * NS + s) * K
                pltpu.sync_copy(ei_hbm.at[1, pl.ds(e0, K)], aidx0)
                pltpu.sync_copy(
                    eh_hbm.at[pl.ds(e0, K), pl.ds(c * H, H)], rowa0)
                pltpu.sync_copy(rowa0, acc.at[aidx0], add=True)

        plsc.subcore_barrier()
        per = n_nodes // NS
        r0 = s * per
        pltpu.sync_copy(acc.at[pl.ds(r0, per)],
                        nh_hbm.at[pl.ds(r0, per), pl.ds(c * H, H)])

    return body(eh, edge_index)


def _tc_layer(eh, em0, em1, W, b, last):
    e, d = eh.shape
    be = 3200
    grid = (e // be,)
    b2 = b.reshape(1, d)

    def mm_body(eh_ref, em0_ref, em1_ref, w_ref, b_ref, *outs):
        w = w_ref[...]
        upd = jnp.dot(em0_ref[...], w[:H, :], preferred_element_type=jnp.float32)
        upd = upd + jnp.dot(em1_ref[...], w[H:, :],
                            preferred_element_type=jnp.float32)
        ehn = eh_ref[...] + upd + b_ref[...]
        outs[0][...] = ehn
        if not last:
            h = jnp.maximum(ehn, 0.0)
            outs[1][...] = h[:, :H]
            outs[2][...] = h[:, H:]

    out_shape = [_f32(e, d)]
    out_specs = [pl.BlockSpec((be, d), lambda i: (i, 0))]
    if not last:
        out_shape += [_f32(e, H), _f32(e, H)]
        out_specs += [pl.BlockSpec((be, H), lambda i: (i, 0))] * 2

    res = pl.pallas_call(
        mm_body,
        grid=grid,
        in_specs=[
            pl.BlockSpec((be, d), lambda i: (i, 0)),
            pl.BlockSpec((be, H), lambda i: (i, 0)),
            pl.BlockSpec((be, H), lambda i: (i, 0)),
            pl.BlockSpec((d, d), lambda i: (0, 0)),
            pl.BlockSpec((1, d), lambda i: (0, 0)),
        ],
        out_specs=out_specs,
        out_shape=out_shape,
    )(eh, em0, em1, W, b2)
    return tuple(res)


def kernel(node_feats, edge_feats, edge_index, rev_index,
           W0, b0, W1, b1, W2, b2):
    n_nodes = node_feats.shape[0]
    eh, h0, h1 = _sc_init(node_feats, edge_feats, edge_index)
    for i, (W, b) in enumerate(((W0, b0), (W1, b1), (W2, b2))):
        em0, em1 = _sc_message(h0, h1, edge_index, rev_index, n_nodes)
        if i < 2:
            eh, h0, h1 = _tc_layer(eh, em0, em1, W, b, last=False)
        else:
            (eh,) = _tc_layer(eh, em0, em1, W, b, last=True)
    node_hiddens = _sc_final(eh, edge_index, n_nodes)
    return (node_hiddens, eh)


# speedup vs baseline: 1.9545x; 1.0189x over previous
import functools

import jax
import jax.numpy as jnp
from jax import lax
from jax.experimental import pallas as pl
from jax.experimental.pallas import tpu as pltpu
from jax.experimental.pallas import tpu_sc as plsc

NC = 2
NS = 16
LN = 16
K = 128
CH = 4
GK = K * CH
H = 64


def _mesh():
    return plsc.VectorSubcoreMesh(core_axis_name="c", subcore_axis_name="s")


_SC_PARAMS = pltpu.CompilerParams(use_tc_tiling_on_sc=False)


def _f32(*shape):
    return jax.ShapeDtypeStruct(shape, jnp.float32)


def _chunk_op(body, j):
    def outer(r, _):
        for q in range(H // LN):
            body(j * K + r, pl.ds(q * LN, LN))
        return _
    lax.fori_loop(0, K, outer, None)


def _zero_acc(s, acc, zbuf, n_rows):
    def z(r, _):
        for q in range(H // LN):
            zbuf[r, pl.ds(q * LN, LN)] = jnp.zeros((LN,), jnp.float32)
        return _
    lax.fori_loop(0, K, z, None)
    per = n_rows // NS
    r0 = s * per
    off = 0
    while off < per:
        kk = min(K, per - off)
        pltpu.sync_copy(zbuf.at[pl.ds(0, kk)], acc.at[pl.ds(r0 + off, kk)])
        off += kk


def _sc_init(node_feats, edge_feats, src2d):
    n, d = node_feats.shape
    ngrp = src2d.shape[0] // CH
    e = src2d.shape[0] * K
    nmain = ngrp // NS
    nleft = ngrp - nmain * NS
    assert nmain % 2 == 1 and nmain >= 5 and n % NS == 0

    @functools.partial(
        pl.kernel,
        out_type=(_f32(e, H), _f32(e, H)),
        mesh=_mesh(),
        compiler_params=_SC_PARAMS,
        scratch_types=[
            pltpu.VMEM_SHARED((n, H), jnp.float32),
            pltpu.VMEM((CH, K), jnp.int32),
            pltpu.VMEM((CH, K), jnp.int32),
            pltpu.VMEM((K, H), jnp.float32),
            pltpu.VMEM((K, H), jnp.float32),
            pltpu.VMEM((GK, H), jnp.float32),
            pltpu.VMEM((GK, H), jnp.float32),
            pltpu.SemaphoreType.DMA,
            pltpu.SemaphoreType.DMA,
            pltpu.SemaphoreType.DMA,
            pltpu.SemaphoreType.DMA,
            pltpu.SemaphoreType.DMA,
            pltpu.SemaphoreType.DMA,
            pltpu.SemaphoreType.DMA,
            pltpu.SemaphoreType.DMA,
        ],
    )
    def body(nf_hbm, ef_hbm, si_hbm, eh0_hbm, eh1_hbm,
             nf_sh, sidxa, sidxb, nfp0, nfp1, efa, efb,
             isem0, isem1, esem0, esem1,
             n0, n1, osem0, osem1):
        c = lax.axis_index("c")
        s = lax.axis_index("s")
        sidx = (sidxa, sidxb)
        nfp = (nfp0, nfp1)
        efx = (efa, efb)
        isem = (isem0, isem1)
        esem = (esem0, esem1)
        nsem = (n0, n1)
        osem = (osem0, osem1)

        per = n // NS
        r0 = s * per
        pltpu.sync_copy(nf_hbm.at[pl.ds(r0, per), pl.ds(c * H, H)],
                        nf_sh.at[pl.ds(r0, per)])
        plsc.subcore_barrier()

        def gid_of(i):
            return s + i * NS

        def start_arr(i, b):
            g = gid_of(i)
            pltpu.async_copy(si_hbm.at[pl.ds(CH * g, CH)], sidx[b], isem[b])
            pltpu.async_copy(
                ef_hbm.at[pl.ds(GK * g, GK), pl.ds(c * H, H)],
                efx[b], esem[b])

        def wait_arr(i, b):
            g = gid_of(i)
            pltpu.make_async_copy(si_hbm.at[pl.ds(CH * g, CH)],
                                  sidx[b], isem[b]).wait()
            pltpu.make_async_copy(
                ef_hbm.at[pl.ds(GK * g, GK), pl.ds(c * H, H)],
                efx[b], esem[b]).wait()

        def wait_out(i, b):
            g = gid_of(i)

            @pl.when(c == 0)
            def _():
                pltpu.make_async_copy(
                    efx[b], eh0_hbm.at[pl.ds(GK * g, GK)], osem[b]).wait()

            @pl.when(c == 1)
            def _():
                pltpu.make_async_copy(
                    efx[b], eh1_hbm.at[pl.ds(GK * g, GK)], osem[b]).wait()

        def gath_adds(b):
            cps = [None] * CH
            cps[0] = pltpu.async_copy(nf_sh.at[sidx[b].at[0]], nfp0, nsem[0])
            for j in range(CH):
                if j + 1 < CH:
                    cps[j + 1] = pltpu.async_copy(
                        nf_sh.at[sidx[b].at[j + 1]],
                        nfp[(j + 1) % 2], nsem[(j + 1) % 2])
                cps[j].wait()
                nfj = nfp[j % 2]

                def op(r, sl):
                    efx[b][j * K + r, sl] = efx[b][j * K + r, sl] + nfj[r, sl]

                def outer(r, _):
                    for q in range(H // LN):
                        op(r, pl.ds(q * LN, LN))
                    return _
                lax.fori_loop(0, K, outer, None)

        def group(i, b, issue_next, wait_o):
            wait_arr(i, b)
            if issue_next:
                if wait_o:
                    wait_out(i - 1, 1 - b)
                start_arr(i + 1, 1 - b)
            gath_adds(b)
            g = gid_of(i)

            @pl.when(c == 0)
            def _():
                pltpu.async_copy(efx[b], eh0_hbm.at[pl.ds(GK * g, GK)],
                                 osem[b])

            @pl.when(c == 1)
            def _():
                pltpu.async_copy(efx[b], eh1_hbm.at[pl.ds(GK * g, GK)],
                                 osem[b])

        start_arr(0, 0)
        group(0, 0, True, False)

        def grp(g, _):
            group(2 * g + 1, 1, True, True)
            group(2 * g + 2, 0, True, True)
            return _
        lax.fori_loop(0, (nmain - 3) // 2, grp, None)
        group(nmain - 2, 1, True, True)
        group(nmain - 1, 0, False, False)
        wait_out(nmain - 2, 1)
        wait_out(nmain - 1, 0)

        if nleft:
            @pl.when(s < nleft)
            def _():
                g = nmain * NS + s
                pltpu.sync_copy(si_hbm.at[pl.ds(CH * g, CH)], sidxa)
                pltpu.sync_copy(
                    ef_hbm.at[pl.ds(GK * g, GK), pl.ds(c * H, H)], efa)
                gath_adds(0)

                @pl.when(c == 0)
                def _():
                    pltpu.sync_copy(efa, eh0_hbm.at[pl.ds(GK * g, GK)])

                @pl.when(c == 1)
                def _():
                    pltpu.sync_copy(efa, eh1_hbm.at[pl.ds(GK * g, GK)])

    return body(node_feats, edge_feats, src2d)


def _sc_message(eh0, eh1, dest2d, src2d, rev2d, n_nodes):
    ngrp = dest2d.shape[0] // CH
    e = dest2d.shape[0] * K
    nmain = ngrp // NS
    nleft = ngrp - nmain * NS
    assert nmain % 2 == 1 and nmain >= 5

    @functools.partial(
        pl.kernel,
        out_type=(_f32(e, H), _f32(e, H)),
        mesh=_mesh(),
        compiler_params=_SC_PARAMS,
        scratch_types=[
            pltpu.VMEM_SHARED((n_nodes, H), jnp.float32),
            pltpu.VMEM((CH, K), jnp.int32),
            pltpu.VMEM((CH, K), jnp.int32),
            pltpu.VMEM((CH, K), jnp.int32),
            pltpu.VMEM((CH, K), jnp.int32),
            pltpu.VMEM((GK, H), jnp.float32),
            pltpu.VMEM((GK, H), jnp.float32),
            pltpu.SemaphoreType.DMA,
            pltpu.SemaphoreType.DMA,
            pltpu.SemaphoreType.DMA,
            pltpu.SemaphoreType.DMA,
            pltpu.SemaphoreType.DMA,
            pltpu.SemaphoreType.DMA,
            pltpu.SemaphoreType.DMA,
            pltpu.SemaphoreType.DMA,
            pltpu.SemaphoreType.DMA,
            pltpu.SemaphoreType.DMA,
            pltpu.SemaphoreType.DMA,
            pltpu.SemaphoreType.DMA,
            pltpu.SemaphoreType.DMA,
            pltpu.SemaphoreType.DMA,
            pltpu.SemaphoreType.DMA,
        ],
    )
    def body(eh0_hbm, eh1_hbm, di_hbm, si_hbm, ri_hbm, em0_hbm, em1_hbm,
             acc, dixa, dixb, rixa, rixb, bufa, bufb,
             isem0, isem1, rsem0, rsem1, hsem0, hsem1,
             n0, n1, n2, n3, e0s, e1s, e2s, e3s, osem):
        c = lax.axis_index("c")
        s = lax.axis_index("s")
        dix = (dixa, dixb)
        rix = (rixa, rixb)
        buf = (bufa, bufb)
        isem = (isem0, isem1)
        rsem = (rsem0, rsem1)
        hsem = (hsem0, hsem1)
        nsem = (n0, n1, n2, n3)
        esem = (e0s, e1s, e2s, e3s)

        def gid_of(i):
            return s + i * NS

        def run(eh_hbm, em_hbm):
            _zero_acc(s, acc, bufa, n_nodes)
            plsc.subcore_barrier()

            def start_arr(i, b):
                g = gid_of(i)
                pltpu.async_copy(di_hbm.at[pl.ds(CH * g, CH)],
                                 dix[b], isem[b])
                pltpu.async_copy(eh_hbm.at[pl.ds(GK * g, GK)],
                                 buf[b], hsem[b])

            def scat_group(i, b, issue_next):
                if issue_next:
                    start_arr(i + 1, 1 - b)
                g = gid_of(i)
                pltpu.make_async_copy(di_hbm.at[pl.ds(CH * g, CH)],
                                      dix[b], isem[b]).wait()
                pltpu.make_async_copy(eh_hbm.at[pl.ds(GK * g, GK)],
                                      buf[b], hsem[b]).wait()
                for j in range(CH):
                    def op(r, sl):
                        buf[b][r, sl] = jnp.maximum(buf[b][r, sl], 0.0)
                    _chunk_op(op, j)
                    pltpu.sync_copy(buf[b].at[pl.ds(j * K, K)],
                                    acc.at[dix[b].at[j]], add=True)

            start_arr(0, 0)
            scat_group(0, 0, True)

            def sgrp(g, _):
                scat_group(2 * g + 1, 1, True)
                scat_group(2 * g + 2, 0, True)
                return _
            lax.fori_loop(0, (nmain - 3) // 2, sgrp, None)
            scat_group(nmain - 2, 1, True)
            scat_group(nmain - 1, 0, False)

            if nleft:
                @pl.when(s < nleft)
                def _():
                    g = nmain * NS + s
                    pltpu.sync_copy(di_hbm.at[pl.ds(CH * g, CH)], dixa)
                    pltpu.sync_copy(eh_hbm.at[pl.ds(GK * g, GK)], bufa)
                    for j in range(CH):
                        def op(r, sl):
                            bufa[r, sl] = jnp.maximum(bufa[r, sl], 0.0)
                        _chunk_op(op, j)
                        pltpu.sync_copy(bufa.at[pl.ds(j * K, K)],
                                        acc.at[dixa.at[j]], add=True)

            plsc.subcore_barrier()

            def start_idx(i, b):
                g = gid_of(i)
                pltpu.async_copy(si_hbm.at[pl.ds(CH * g, CH)],
                                 dix[b], isem[b])
                pltpu.async_copy(ri_hbm.at[pl.ds(CH * g, CH)],
                                 rix[b], rsem[b])

            def wait_em_out(i):
                g = gid_of(i)
                pltpu.make_async_copy(
                    bufa, em_hbm.at[pl.ds(GK * g, GK)], osem).wait()

            def gath_group(i, b, issue_idx, drain_out):
                g = gid_of(i)
                if drain_out:
                    wait_em_out(i - 1)
                pltpu.make_async_copy(si_hbm.at[pl.ds(CH * g, CH)],
                                      dix[b], isem[b]).wait()
                pltpu.make_async_copy(ri_hbm.at[pl.ds(CH * g, CH)],
                                      rix[b], rsem[b]).wait()
                cpn = [pltpu.async_copy(acc.at[dix[b].at[j]],
                                        bufa.at[pl.ds(j * K, K)], nsem[j])
                       for j in range(CH)]
                cpe = [pltpu.async_copy(eh_hbm.at[rix[b].at[j]],
                                        bufb.at[pl.ds(j * K, K)], esem[j])
                       for j in range(CH)]
                if issue_idx:
                    start_idx(i + 1, 1 - b)
                for j in range(CH):
                    cpn[j].wait()
                    cpe[j].wait()

                    def op(r, sl):
                        bufa[r, sl] = bufa[r, sl] - jnp.maximum(
                            bufb[r, sl], 0.0)
                    _chunk_op(op, j)
                pltpu.async_copy(bufa, em_hbm.at[pl.ds(GK * g, GK)], osem)

            start_idx(0, 0)
            gath_group(0, 0, True, False)

            def ggrp(g, _):
                gath_group(2 * g + 1, 1, True, True)
                gath_group(2 * g + 2, 0, True, True)
                return _
            lax.fori_loop(0, (nmain - 3) // 2, ggrp, None)
            gath_group(nmain - 2, 1, True, True)
            gath_group(nmain - 1, 0, False, True)
            wait_em_out(nmain - 1)

            if nleft:
                @pl.when(s < nleft)
                def _():
                    g = nmain * NS + s
                    pltpu.sync_copy(si_hbm.at[pl.ds(CH * g, CH)], dixa)
                    pltpu.sync_copy(ri_hbm.at[pl.ds(CH * g, CH)], rixa)
                    cpn = [pltpu.async_copy(acc.at[dixa.at[j]],
                                            bufa.at[pl.ds(j * K, K)],
                                            nsem[j])
                           for j in range(CH)]
                    cpe = [pltpu.async_copy(eh_hbm.at[rixa.at[j]],
                                            bufb.at[pl.ds(j * K, K)],
                                            esem[j])
                           for j in range(CH)]
                    for j in range(CH):
                        cpn[j].wait()
                        cpe[j].wait()

                        def op(r, sl):
                            bufa[r, sl] = bufa[r, sl] - jnp.maximum(
                                bufb[r, sl], 0.0)
                        _chunk_op(op, j)
                    pltpu.sync_copy(bufa, em_hbm.at[pl.ds(GK * g, GK)])

        @pl.when(c == 0)
        def _():
            run(eh0_hbm, em0_hbm)

        @pl.when(c == 1)
        def _():
            run(eh1_hbm, em1_hbm)

    return body(eh0, eh1, dest2d, src2d, rev2d)


def _sc_final(eh, dest2d, n_nodes):
    e, d = eh.shape
    ngrp = dest2d.shape[0] // CH
    nmain = ngrp // NS
    nleft = ngrp - nmain * NS
    assert nmain % 2 == 1 and nmain >= 5

    @functools.partial(
        pl.kernel,
        out_type=_f32(n_nodes, d),
        mesh=_mesh(),
        compiler_params=_SC_PARAMS,
        scratch_types=[
            pltpu.VMEM_SHARED((n_nodes, H), jnp.float32),
            pltpu.VMEM((CH, K), jnp.int32),
            pltpu.VMEM((CH, K), jnp.int32),
            pltpu.VMEM((GK, H), jnp.float32),
            pltpu.VMEM((GK, H), jnp.float32),
            pltpu.SemaphoreType.DMA,
            pltpu.SemaphoreType.DMA,
            pltpu.SemaphoreType.DMA,
            pltpu.SemaphoreType.DMA,
        ],
    )
    def body(eh_hbm, di_hbm, nh_hbm,
             acc, dixa, dixb, bufa, bufb, isem0, isem1, hsem0, hsem1):
        c = lax.axis_index("c")
        s = lax.axis_index("s")
        dix = (dixa, dixb)
        buf = (bufa, bufb)
        isem = (isem0, isem1)
        hsem = (hsem0, hsem1)
        _zero_acc(s, acc, bufa, n_nodes)
        plsc.subcore_barrier()

        def gid_of(i):
            return s + i * NS

        def start_arr(i, b):
            g = gid_of(i)
            pltpu.async_copy(di_hbm.at[pl.ds(CH * g, CH)], dix[b], isem[b])
            pltpu.async_copy(eh_hbm.at[pl.ds(GK * g, GK), pl.ds(c * H, H)],
                             buf[b], hsem[b])

        def scat_group(i, b, issue_next):
            if issue_next:
                start_arr(i + 1, 1 - b)
            g = gid_of(i)
            pltpu.make_async_copy(di_hbm.at[pl.ds(CH * g, CH)],
                                  dix[b], isem[b]).wait()
            pltpu.make_async_copy(
                eh_hbm.at[pl.ds(GK * g, GK), pl.ds(c * H, H)],
                buf[b], hsem[b]).wait()
            for j in range(CH):
                pltpu.sync_copy(buf[b].at[pl.ds(j * K, K)],
                                acc.at[dix[b].at[j]], add=True)

        start_arr(0, 0)
        scat_group(0, 0, True)

        def sgrp(g, _):
            scat_group(2 * g + 1, 1, True)
            scat_group(2 * g + 2, 0, True)
            return _
        lax.fori_loop(0, (nmain - 3) // 2, sgrp, None)
        scat_group(nmain - 2, 1, True)
        scat_group(nmain - 1, 0, False)

        if nleft:
            @pl.when(s < nleft)
            def _():
                g = nmain * NS + s
                pltpu.sync_copy(di_hbm.at[pl.ds(CH * g, CH)], dixa)
                pltpu.sync_copy(
                    eh_hbm.at[pl.ds(GK * g, GK), pl.ds(c * H, H)], bufa)
                for j in range(CH):
                    pltpu.sync_copy(bufa.at[pl.ds(j * K, K)],
                                    acc.at[dixa.at[j]], add=True)

        plsc.subcore_barrier()
        per = n_nodes // NS
        r0 = s * per
        pltpu.sync_copy(acc.at[pl.ds(r0, per)],
                        nh_hbm.at[pl.ds(r0, per), pl.ds(c * H, H)])

    return body(eh, dest2d)


def _tc_layer(eh0, eh1, em0, em1, W, b, last):
    e = eh0.shape[0]
    d = W.shape[0]
    be = 3200
    grid = (e // be,)
    b2 = b.reshape(1, d)

    def mm_body(eh0_ref, eh1_ref, em0_ref, em1_ref, w_ref, b_ref, *outs):
        w = w_ref[...]
        upd = jnp.dot(em0_ref[...], w[:H, :],
                      preferred_element_type=jnp.float32)
        upd = upd + jnp.dot(em1_ref[...], w[H:, :],
                            preferred_element_type=jnp.float32)
        upd = upd + b_ref[...]
        if last:
            outs[0][:, :H] = eh0_ref[...] + upd[:, :H]
            outs[0][:, H:] = eh1_ref[...] + upd[:, H:]
        else:
            outs[0][...] = eh0_ref[...] + upd[:, :H]
            outs[1][...] = eh1_ref[...] + upd[:, H:]

    if last:
        out_shape = [_f32(e, d)]
        out_specs = [pl.BlockSpec((be, d), lambda i: (i, 0))]
    else:
        out_shape = [_f32(e, H), _f32(e, H)]
        out_specs = [pl.BlockSpec((be, H), lambda i: (i, 0))] * 2

    res = pl.pallas_call(
        mm_body,
        grid=grid,
        in_specs=[
            pl.BlockSpec((be, H), lambda i: (i, 0)),
            pl.BlockSpec((be, H), lambda i: (i, 0)),
            pl.BlockSpec((be, H), lambda i: (i, 0)),
            pl.BlockSpec((be, H), lambda i: (i, 0)),
            pl.BlockSpec((d, d), lambda i: (0, 0)),
            pl.BlockSpec((1, d), lambda i: (0, 0)),
        ],
        out_specs=out_specs,
        out_shape=out_shape,
    )(eh0, eh1, em0, em1, W, b2)
    return tuple(res)


def kernel(node_feats, edge_feats, edge_index, rev_index,
           W0, b0, W1, b1, W2, b2):
    n_nodes = node_feats.shape[0]
    e = edge_feats.shape[0]
    src2d = edge_index[0].reshape(e // K, K)
    dest2d = edge_index[1].reshape(e // K, K)
    rev2d = rev_index.reshape(e // K, K)
    eh0, eh1 = _sc_init(node_feats, edge_feats, src2d)
    for i, (W, b) in enumerate(((W0, b0), (W1, b1), (W2, b2))):
        em0, em1 = _sc_message(eh0, eh1, dest2d, src2d, rev2d, n_nodes)
        if i < 2:
            eh0, eh1 = _tc_layer(eh0, eh1, em0, em1, W, b, last=False)
        else:
            (eh,) = _tc_layer(eh0, eh1, em0, em1, W, b, last=True)
    node_hiddens = _sc_final(eh, dest2d, n_nodes)
    return (node_hiddens, eh)


# speedup vs baseline: 2.3584x; 1.2066x over previous
import functools

import jax
import jax.numpy as jnp
from jax import lax
from jax.experimental import pallas as pl
from jax.experimental.pallas import tpu as pltpu
from jax.experimental.pallas import tpu_sc as plsc

NC = 2
NS = 16
LN = 16
K = 128
CH = 4
GK = K * CH
H = 64
D = 2 * H


def _mesh():
    return plsc.VectorSubcoreMesh(core_axis_name="c", subcore_axis_name="s")


_SC_PARAMS = pltpu.CompilerParams(use_tc_tiling_on_sc=False)


def _f32(*shape):
    return jax.ShapeDtypeStruct(shape, jnp.float32)


def _chunk_op(body, j):
    def outer(r, _):
        for q in range(H // LN):
            body(j * K + r, pl.ds(q * LN, LN))
        return _
    lax.fori_loop(0, K, outer, None)


def _zero_acc(s, acc, zbuf, n_rows):
    def z(r, _):
        for q in range(H // LN):
            zbuf[r, pl.ds(q * LN, LN)] = jnp.zeros((LN,), jnp.float32)
        return _
    lax.fori_loop(0, K, z, None)
    per = n_rows // NS
    r0 = s * per
    off = 0
    while off < per:
        kk = min(K, per - off)
        pltpu.sync_copy(zbuf.at[pl.ds(0, kk)], acc.at[pl.ds(r0 + off, kk)])
        off += kk


def _sc_init(node_feats, edge_feats, src2d):
    n, d = node_feats.shape
    ngrp = src2d.shape[0] // CH
    e = src2d.shape[0] * K
    nmain = ngrp // NS
    nleft = ngrp - nmain * NS
    assert nmain % 2 == 1 and nmain >= 5 and n % NS == 0

    @functools.partial(
        pl.kernel,
        out_type=_f32(e, D),
        mesh=_mesh(),
        compiler_params=_SC_PARAMS,
        scratch_types=[
            pltpu.VMEM_SHARED((n, H), jnp.float32),
            pltpu.VMEM((CH, K), jnp.int32),
            pltpu.VMEM((CH, K), jnp.int32),
            pltpu.VMEM((K, H), jnp.float32),
            pltpu.VMEM((K, H), jnp.float32),
            pltpu.VMEM((GK, H), jnp.float32),
            pltpu.VMEM((GK, H), jnp.float32),
            pltpu.SemaphoreType.DMA,
            pltpu.SemaphoreType.DMA,
            pltpu.SemaphoreType.DMA,
            pltpu.SemaphoreType.DMA,
            pltpu.SemaphoreType.DMA,
            pltpu.SemaphoreType.DMA,
            pltpu.SemaphoreType.DMA,
            pltpu.SemaphoreType.DMA,
        ],
    )
    def body(nf_hbm, ef_hbm, si_hbm, eh_hbm,
             nf_sh, sidxa, sidxb, nfp0, nfp1, efa, efb,
             isem0, isem1, esem0, esem1,
             n0, n1, osem0, osem1):
        c = lax.axis_index("c")
        s = lax.axis_index("s")
        sidx = (sidxa, sidxb)
        nfp = (nfp0, nfp1)
        efx = (efa, efb)
        isem = (isem0, isem1)
        esem = (esem0, esem1)
        nsem = (n0, n1)
        osem = (osem0, osem1)
        cs = pl.ds(c * H, H)

        per = n // NS
        r0 = s * per
        pltpu.sync_copy(nf_hbm.at[pl.ds(r0, per), cs],
                        nf_sh.at[pl.ds(r0, per)])
        plsc.subcore_barrier()

        def gid_of(i):
            return s + i * NS

        def start_arr(i, b):
            g = gid_of(i)
            pltpu.async_copy(si_hbm.at[pl.ds(CH * g, CH)], sidx[b], isem[b])
            pltpu.async_copy(ef_hbm.at[pl.ds(GK * g, GK), cs],
                             efx[b], esem[b])

        def wait_arr(i, b):
            g = gid_of(i)
            pltpu.make_async_copy(si_hbm.at[pl.ds(CH * g, CH)],
                                  sidx[b], isem[b]).wait()
            pltpu.make_async_copy(ef_hbm.at[pl.ds(GK * g, GK), cs],
                                  efx[b], esem[b]).wait()

        def wait_out(i, b):
            g = gid_of(i)
            pltpu.make_async_copy(
                efx[b], eh_hbm.at[pl.ds(GK * g, GK), cs], osem[b]).wait()

        def gath_adds(b):
            cps = [None] * CH
            cps[0] = pltpu.async_copy(nf_sh.at[sidx[b].at[0]], nfp0, nsem[0])
            for j in range(CH):
                if j + 1 < CH:
                    cps[j + 1] = pltpu.async_copy(
                        nf_sh.at[sidx[b].at[j + 1]],
                        nfp[(j + 1) % 2], nsem[(j + 1) % 2])
                cps[j].wait()
                nfj = nfp[j % 2]

                def op(r, sl):
                    efx[b][j * K + r, sl] = efx[b][j * K + r, sl] + nfj[r, sl]

                def outer(r, _):
                    for q in range(H // LN):
                        op(r, pl.ds(q * LN, LN))
                    return _
                lax.fori_loop(0, K, outer, None)

        def group(i, b, issue_next, wait_o):
            wait_arr(i, b)
            if issue_next:
                if wait_o:
                    wait_out(i - 1, 1 - b)
                start_arr(i + 1, 1 - b)
            gath_adds(b)
            g = gid_of(i)
            pltpu.async_copy(efx[b], eh_hbm.at[pl.ds(GK * g, GK), cs],
                             osem[b])

        start_arr(0, 0)
        group(0, 0, True, False)

        def grp(g, _):
            group(2 * g + 1, 1, True, True)
            group(2 * g + 2, 0, True, True)
            return _
        lax.fori_loop(0, (nmain - 3) // 2, grp, None)
        group(nmain - 2, 1, True, True)
        group(nmain - 1, 0, False, False)
        wait_out(nmain - 2, 1)
        wait_out(nmain - 1, 0)

        if nleft:
            @pl.when(s < nleft)
            def _():
                g = nmain * NS + s
                pltpu.sync_copy(si_hbm.at[pl.ds(CH * g, CH)], sidxa)
                pltpu.sync_copy(ef_hbm.at[pl.ds(GK * g, GK), cs], efa)
                gath_adds(0)
                pltpu.sync_copy(efa, eh_hbm.at[pl.ds(GK * g, GK), cs])

    return body(node_feats, edge_feats, src2d)


def _sc_message(eh, dest2d, src2d, rev2d, n_nodes):
    ngrp = dest2d.shape[0] // CH
    e = dest2d.shape[0] * K
    nmain = ngrp // NS
    nleft = ngrp - nmain * NS
    assert nmain % 2 == 1 and nmain >= 5

    @functools.partial(
        pl.kernel,
        out_type=_f32(e, D),
        mesh=_mesh(),
        compiler_params=_SC_PARAMS,
        scratch_types=[
            pltpu.VMEM_SHARED((n_nodes, H), jnp.float32),
            pltpu.VMEM((CH, K), jnp.int32),
            pltpu.VMEM((CH, K), jnp.int32),
            pltpu.VMEM((CH, K), jnp.int32),
            pltpu.VMEM((CH, K), jnp.int32),
            pltpu.VMEM((GK, H), jnp.float32),
            pltpu.VMEM((GK, H), jnp.float32),
            pltpu.VMEM((K // 2, D), jnp.float32),
            pltpu.VMEM((K // 2, D), jnp.float32),
            pltpu.SemaphoreType.DMA,
            pltpu.SemaphoreType.DMA,
            pltpu.SemaphoreType.DMA,
            pltpu.SemaphoreType.DMA,
            pltpu.SemaphoreType.DMA,
            pltpu.SemaphoreType.DMA,
            pltpu.SemaphoreType.DMA,
            pltpu.SemaphoreType.DMA,
            pltpu.SemaphoreType.DMA,
            pltpu.SemaphoreType.DMA,
            pltpu.SemaphoreType.DMA,
            pltpu.SemaphoreType.DMA,
            pltpu.SemaphoreType.DMA,
        ],
    )
    def body(eh_hbm, di_hbm, si_hbm, ri_hbm, em_hbm,
             acc, dixa, dixb, rixa, rixb, bufa, bufs, reva, revb,
             isem0, isem1, rsem0, rsem1, hsem0, hsem1,
             n0, n1, n2, n3, e0s, e1s, osem):
        c = lax.axis_index("c")
        s = lax.axis_index("s")
        dix = (dixa, dixb)
        rix = (rixa, rixb)
        buf = (bufa, bufs)
        rev = (reva, revb)
        isem = (isem0, isem1)
        rsem = (rsem0, rsem1)
        hsem = (hsem0, hsem1)
        nsem = (n0, n1, n2, n3)
        esem = (e0s, e1s)
        cs = pl.ds(c * H, H)

        def gid_of(i):
            return s + i * NS

        _zero_acc(s, acc, bufa, n_nodes)
        plsc.subcore_barrier()

        def start_arr(i, b):
            g = gid_of(i)
            pltpu.async_copy(di_hbm.at[pl.ds(CH * g, CH)], dix[b], isem[b])
            pltpu.async_copy(eh_hbm.at[pl.ds(GK * g, GK), cs],
                             buf[b], hsem[b])

        def scat_group(i, b, issue_next):
            if issue_next:
                start_arr(i + 1, 1 - b)
            g = gid_of(i)
            pltpu.make_async_copy(di_hbm.at[pl.ds(CH * g, CH)],
                                  dix[b], isem[b]).wait()
            pltpu.make_async_copy(eh_hbm.at[pl.ds(GK * g, GK), cs],
                                  buf[b], hsem[b]).wait()
            for j in range(CH):
                def op(r, sl):
                    buf[b][r, sl] = jnp.maximum(buf[b][r, sl], 0.0)
                _chunk_op(op, j)
                pltpu.sync_copy(buf[b].at[pl.ds(j * K, K)],
                                acc.at[dix[b].at[j]], add=True)

        start_arr(0, 0)
        scat_group(0, 0, True)

        def sgrp(g, _):
            scat_group(2 * g + 1, 1, True)
            scat_group(2 * g + 2, 0, True)
            return _
        lax.fori_loop(0, (nmain - 3) // 2, sgrp, None)
        scat_group(nmain - 2, 1, True)
        scat_group(nmain - 1, 0, False)

        if nleft:
            @pl.when(s < nleft)
            def _():
                g = nmain * NS + s
                pltpu.sync_copy(di_hbm.at[pl.ds(CH * g, CH)], dixa)
                pltpu.sync_copy(eh_hbm.at[pl.ds(GK * g, GK), cs], bufa)
                for j in range(CH):
                    def op(r, sl):
                        bufa[r, sl] = jnp.maximum(bufa[r, sl], 0.0)
                    _chunk_op(op, j)
                    pltpu.sync_copy(bufa.at[pl.ds(j * K, K)],
                                    acc.at[dixa.at[j]], add=True)

        plsc.subcore_barrier()

        def start_idx(i, b):
            g = gid_of(i)
            pltpu.async_copy(si_hbm.at[pl.ds(CH * g, CH)], dix[b], isem[b])
            pltpu.async_copy(ri_hbm.at[pl.ds(CH * g, CH)], rix[b], rsem[b])

        def wait_em_out(i):
            g = gid_of(i)
            pltpu.make_async_copy(
                bufa, em_hbm.at[pl.ds(GK * g, GK), cs], osem).wait()

        KH = K // 2
        T = 2 * CH

        def rev_sub(t):
            rv = rev[t % 2]
            base = t * KH

            def outer(r, _):
                for q in range(H // LN):
                    sl = pl.ds(q * LN, LN)
                    sl2 = pl.ds(c * H + q * LN, LN)
                    bufa[base + r, sl] = bufa[base + r, sl] - jnp.maximum(
                        rv[r, sl2], 0.0)
                return _
            lax.fori_loop(0, KH, outer, None)

        def issue_rev(b, t):
            j, u = divmod(t, 2)
            return pltpu.async_copy(
                eh_hbm.at[rix[b].at[j, pl.ds(u * KH, KH)]],
                rev[t % 2], esem[t % 2])

        def gath_body(b):
            cpn = [pltpu.async_copy(acc.at[dix[b].at[j]],
                                    bufa.at[pl.ds(j * K, K)], nsem[j])
                   for j in range(CH)]
            cpe = [None] * T
            cpe[0] = issue_rev(b, 0)
            cpe[1] = issue_rev(b, 1)
            for t in range(T):
                if t % 2 == 0:
                    cpn[t // 2].wait()
                cpe[t].wait()
                rev_sub(t)
                if t + 2 < T:
                    cpe[t + 2] = issue_rev(b, t + 2)

        def gath_group(i, b, issue_idx, drain_out):
            g = gid_of(i)
            if drain_out:
                wait_em_out(i - 1)
            pltpu.make_async_copy(si_hbm.at[pl.ds(CH * g, CH)],
                                  dix[b], isem[b]).wait()
            pltpu.make_async_copy(ri_hbm.at[pl.ds(CH * g, CH)],
                                  rix[b], rsem[b]).wait()
            if issue_idx:
                start_idx(i + 1, 1 - b)
            gath_body(b)
            pltpu.async_copy(bufa, em_hbm.at[pl.ds(GK * g, GK), cs], osem)

        start_idx(0, 0)
        gath_group(0, 0, True, False)

        def ggrp(g, _):
            gath_group(2 * g + 1, 1, True, True)
            gath_group(2 * g + 2, 0, True, True)
            return _
        lax.fori_loop(0, (nmain - 3) // 2, ggrp, None)
        gath_group(nmain - 2, 1, True, True)
        gath_group(nmain - 1, 0, False, True)
        wait_em_out(nmain - 1)

        if nleft:
            @pl.when(s < nleft)
            def _():
                g = nmain * NS + s
                pltpu.sync_copy(si_hbm.at[pl.ds(CH * g, CH)], dixa)
                pltpu.sync_copy(ri_hbm.at[pl.ds(CH * g, CH)], rixa)
                gath_body(0)
                pltpu.sync_copy(bufa, em_hbm.at[pl.ds(GK * g, GK), cs])

    return body(eh, dest2d, src2d, rev2d)


def _sc_final(eh, dest2d, n_nodes):
    e, d = eh.shape
    ngrp = dest2d.shape[0] // CH
    nmain = ngrp // NS
    nleft = ngrp - nmain * NS
    assert nmain % 2 == 1 and nmain >= 5

    @functools.partial(
        pl.kernel,
        out_type=_f32(n_nodes, d),
        mesh=_mesh(),
        compiler_params=_SC_PARAMS,
        scratch_types=[
            pltpu.VMEM_SHARED((n_nodes, H), jnp.float32),
            pltpu.VMEM((CH, K), jnp.int32),
            pltpu.VMEM((CH, K), jnp.int32),
            pltpu.VMEM((GK, H), jnp.float32),
            pltpu.VMEM((GK, H), jnp.float32),
            pltpu.SemaphoreType.DMA,
            pltpu.SemaphoreType.DMA,
            pltpu.SemaphoreType.DMA,
            pltpu.SemaphoreType.DMA,
        ],
    )
    def body(eh_hbm, di_hbm, nh_hbm,
             acc, dixa, dixb, bufa, bufb, isem0, isem1, hsem0, hsem1):
        c = lax.axis_index("c")
        s = lax.axis_index("s")
        dix = (dixa, dixb)
        buf = (bufa, bufb)
        isem = (isem0, isem1)
        hsem = (hsem0, hsem1)
        _zero_acc(s, acc, bufa, n_nodes)
        plsc.subcore_barrier()

        def gid_of(i):
            return s + i * NS

        def start_arr(i, b):
            g = gid_of(i)
            pltpu.async_copy(di_hbm.at[pl.ds(CH * g, CH)], dix[b], isem[b])
            pltpu.async_copy(eh_hbm.at[pl.ds(GK * g, GK), pl.ds(c * H, H)],
                             buf[b], hsem[b])

        def scat_group(i, b, issue_next):
            if issue_next:
                start_arr(i + 1, 1 - b)
            g = gid_of(i)
            pltpu.make_async_copy(di_hbm.at[pl.ds(CH * g, CH)],
                                  dix[b], isem[b]).wait()
            pltpu.make_async_copy(
                eh_hbm.at[pl.ds(GK * g, GK), pl.ds(c * H, H)],
                buf[b], hsem[b]).wait()
            for j in range(CH):
                pltpu.sync_copy(buf[b].at[pl.ds(j * K, K)],
                                acc.at[dix[b].at[j]], add=True)

        start_arr(0, 0)
        scat_group(0, 0, True)

        def sgrp(g, _):
            scat_group(2 * g + 1, 1, True)
            scat_group(2 * g + 2, 0, True)
            return _
        lax.fori_loop(0, (nmain - 3) // 2, sgrp, None)
        scat_group(nmain - 2, 1, True)
        scat_group(nmain - 1, 0, False)

        if nleft:
            @pl.when(s < nleft)
            def _():
                g = nmain * NS + s
                pltpu.sync_copy(di_hbm.at[pl.ds(CH * g, CH)], dixa)
                pltpu.sync_copy(
                    eh_hbm.at[pl.ds(GK * g, GK), pl.ds(c * H, H)], bufa)
                for j in range(CH):
                    pltpu.sync_copy(bufa.at[pl.ds(j * K, K)],
                                    acc.at[dixa.at[j]], add=True)

        plsc.subcore_barrier()
        per = n_nodes // NS
        r0 = s * per
        pltpu.sync_copy(acc.at[pl.ds(r0, per)],
                        nh_hbm.at[pl.ds(r0, per), pl.ds(c * H, H)])

    return body(eh, dest2d)


def _tc_layer(eh, em, W, b):
    e, d = eh.shape
    be = 3200
    grid = (e // be,)
    b2 = b.reshape(1, d)

    def mm_body(eh_ref, em_ref, w_ref, b_ref, out_ref):
        upd = jnp.dot(em_ref[...], w_ref[...],
                      preferred_element_type=jnp.float32)
        out_ref[...] = eh_ref[...] + upd + b_ref[...]

    return pl.pallas_call(
        mm_body,
        grid=grid,
        in_specs=[
            pl.BlockSpec((be, d), lambda i: (i, 0)),
            pl.BlockSpec((be, d), lambda i: (i, 0)),
            pl.BlockSpec((d, d), lambda i: (0, 0)),
            pl.BlockSpec((1, d), lambda i: (0, 0)),
        ],
        out_specs=pl.BlockSpec((be, d), lambda i: (i, 0)),
        out_shape=_f32(e, d),
    )(eh, em, W, b2)


def kernel(node_feats, edge_feats, edge_index, rev_index,
           W0, b0, W1, b1, W2, b2):
    n_nodes = node_feats.shape[0]
    e = edge_feats.shape[0]
    src2d = edge_index[0].reshape(e // K, K)
    dest2d = edge_index[1].reshape(e // K, K)
    rev2d = rev_index.reshape(e // K, K)
    eh = _sc_init(node_feats, edge_feats, src2d)
    for W, b in ((W0, b0), (W1, b1), (W2, b2)):
        em = _sc_message(eh, dest2d, src2d, rev2d, n_nodes)
        eh = _tc_layer(eh, em, W, b)
    node_hiddens = _sc_final(eh, dest2d, n_nodes)
    return (node_hiddens, eh)


# speedup vs baseline: 3.3456x; 1.4186x over previous
import functools

import jax
import jax.numpy as jnp
from jax import lax
from jax.experimental import pallas as pl
from jax.experimental.pallas import tpu as pltpu
from jax.experimental.pallas import tpu_sc as plsc

NC = 2
NS = 16
LN = 16
K = 128
CH = 4
GK = K * CH
H = 64
D = 2 * H


def _mesh():
    return plsc.VectorSubcoreMesh(core_axis_name="c", subcore_axis_name="s")


_SC_PARAMS = pltpu.CompilerParams(use_tc_tiling_on_sc=False)


def _f32(*shape):
    return jax.ShapeDtypeStruct(shape, jnp.float32)


def _chunk_op(body, j):
    def outer(r, _):
        for q in range(H // LN):
            body(j * K + r, pl.ds(q * LN, LN))
        return _
    lax.fori_loop(0, K, outer, None)


def _zero_acc(s, acc, zbuf, n_rows):
    def z(r, _):
        for q in range(H // LN):
            zbuf[r, pl.ds(q * LN, LN)] = jnp.zeros((LN,), jnp.float32)
        return _
    lax.fori_loop(0, K, z, None)
    per = n_rows // NS
    r0 = s * per
    off = 0
    while off < per:
        kk = min(K, per - off)
        pltpu.sync_copy(zbuf.at[pl.ds(0, kk)], acc.at[pl.ds(r0 + off, kk)])
        off += kk


def _sc_init(node_feats, edge_feats, src2d):
    n, d = node_feats.shape
    ngrp = src2d.shape[0] // CH
    e = src2d.shape[0] * K
    nmain = ngrp // NS
    nleft = ngrp - nmain * NS
    assert nmain % 2 == 1 and nmain >= 5 and n % NS == 0

    @functools.partial(
        pl.kernel,
        out_type=_f32(e, D),
        mesh=_mesh(),
        compiler_params=_SC_PARAMS,
        scratch_types=[
            pltpu.VMEM_SHARED((n, H), jnp.float32),
            pltpu.VMEM((CH, K), jnp.int32),
            pltpu.VMEM((CH, K), jnp.int32),
            pltpu.VMEM((K, H), jnp.float32),
            pltpu.VMEM((K, H), jnp.float32),
            pltpu.VMEM((GK, H), jnp.float32),
            pltpu.VMEM((GK, H), jnp.float32),
            pltpu.SemaphoreType.DMA,
            pltpu.SemaphoreType.DMA,
            pltpu.SemaphoreType.DMA,
            pltpu.SemaphoreType.DMA,
            pltpu.SemaphoreType.DMA,
            pltpu.SemaphoreType.DMA,
            pltpu.SemaphoreType.DMA,
            pltpu.SemaphoreType.DMA,
        ],
    )
    def body(nf_hbm, ef_hbm, si_hbm, eh_hbm,
             nf_sh, sidxa, sidxb, nfp0, nfp1, efa, efb,
             isem0, isem1, esem0, esem1,
             n0, n1, osem0, osem1):
        c = lax.axis_index("c")
        s = lax.axis_index("s")
        sidx = (sidxa, sidxb)
        nfp = (nfp0, nfp1)
        efx = (efa, efb)
        isem = (isem0, isem1)
        esem = (esem0, esem1)
        nsem = (n0, n1)
        osem = (osem0, osem1)
        cs = pl.ds(c * H, H)

        per = n // NS
        r0 = s * per
        pltpu.sync_copy(nf_hbm.at[pl.ds(r0, per), cs],
                        nf_sh.at[pl.ds(r0, per)])
        plsc.subcore_barrier()

        def gid_of(i):
            return s + i * NS

        def start_arr(i, b):
            g = gid_of(i)
            pltpu.async_copy(si_hbm.at[pl.ds(CH * g, CH)], sidx[b], isem[b])
            pltpu.async_copy(ef_hbm.at[pl.ds(GK * g, GK), cs],
                             efx[b], esem[b])

        def wait_arr(i, b):
            g = gid_of(i)
            pltpu.make_async_copy(si_hbm.at[pl.ds(CH * g, CH)],
                                  sidx[b], isem[b]).wait()
            pltpu.make_async_copy(ef_hbm.at[pl.ds(GK * g, GK), cs],
                                  efx[b], esem[b]).wait()

        def wait_out(i, b):
            g = gid_of(i)
            pltpu.make_async_copy(
                efx[b], eh_hbm.at[pl.ds(GK * g, GK), cs], osem[b]).wait()

        def gath_adds(b):
            cps = [None] * CH
            cps[0] = pltpu.async_copy(nf_sh.at[sidx[b].at[0]], nfp0, nsem[0])
            for j in range(CH):
                if j + 1 < CH:
                    cps[j + 1] = pltpu.async_copy(
                        nf_sh.at[sidx[b].at[j + 1]],
                        nfp[(j + 1) % 2], nsem[(j + 1) % 2])
                cps[j].wait()
                nfj = nfp[j % 2]

                def op(r, sl):
                    efx[b][j * K + r, sl] = efx[b][j * K + r, sl] + nfj[r, sl]

                def outer(r, _):
                    for q in range(H // LN):
                        op(r, pl.ds(q * LN, LN))
                    return _
                lax.fori_loop(0, K, outer, None)

        def group(i, b, issue_next, wait_o):
            wait_arr(i, b)
            if issue_next:
                if wait_o:
                    wait_out(i - 1, 1 - b)
                start_arr(i + 1, 1 - b)
            gath_adds(b)
            g = gid_of(i)
            pltpu.async_copy(efx[b], eh_hbm.at[pl.ds(GK * g, GK), cs],
                             osem[b])

        start_arr(0, 0)
        group(0, 0, True, False)

        def grp(g, _):
            group(2 * g + 1, 1, True, True)
            group(2 * g + 2, 0, True, True)
            return _
        lax.fori_loop(0, (nmain - 3) // 2, grp, None)
        group(nmain - 2, 1, True, True)
        group(nmain - 1, 0, False, False)
        wait_out(nmain - 2, 1)
        wait_out(nmain - 1, 0)

        if nleft:
            @pl.when(s < nleft)
            def _():
                g = nmain * NS + s
                pltpu.sync_copy(si_hbm.at[pl.ds(CH * g, CH)], sidxa)
                pltpu.sync_copy(ef_hbm.at[pl.ds(GK * g, GK), cs], efa)
                gath_adds(0)
                pltpu.sync_copy(efa, eh_hbm.at[pl.ds(GK * g, GK), cs])

    return body(node_feats, edge_feats, src2d)


def _sc_message(eh, dest2d, src2d, rev2d, n_nodes):
    ngrp = dest2d.shape[0] // CH
    e = dest2d.shape[0] * K
    nmain = ngrp // NS
    nleft = ngrp - nmain * NS
    assert nmain % 2 == 1 and nmain >= 5

    @functools.partial(
        pl.kernel,
        out_type=(_f32(e, D), _f32(NC, e, H)),
        mesh=_mesh(),
        compiler_params=_SC_PARAMS,
        scratch_types=[
            pltpu.VMEM_SHARED((n_nodes, H), jnp.float32),
            pltpu.VMEM((CH, K), jnp.int32),
            pltpu.VMEM((CH, K), jnp.int32),
            pltpu.VMEM((CH, K), jnp.int32),
            pltpu.VMEM((CH, K), jnp.int32),
            pltpu.VMEM((GK, H), jnp.float32),
            pltpu.VMEM((GK, H), jnp.float32),
            pltpu.SemaphoreType.DMA,
            pltpu.SemaphoreType.DMA,
            pltpu.SemaphoreType.DMA,
            pltpu.SemaphoreType.DMA,
            pltpu.SemaphoreType.DMA,
            pltpu.SemaphoreType.DMA,
            pltpu.SemaphoreType.DMA,
            pltpu.SemaphoreType.DMA,
            pltpu.SemaphoreType.DMA,
            pltpu.SemaphoreType.DMA,
            pltpu.SemaphoreType.DMA,
            pltpu.SemaphoreType.DMA,
            pltpu.SemaphoreType.DMA,
            pltpu.SemaphoreType.DMA,
            pltpu.SemaphoreType.DMA,
            pltpu.SemaphoreType.DMA,
            pltpu.SemaphoreType.DMA,
        ],
    )
    def body(eh_hbm, di_hbm, si_hbm, ri_hbm, em_hbm, h_hbm,
             acc, dixa, dixb, rixa, rixb, bufa, bufs,
             isem0, isem1, rsem0, rsem1, hsem0, hsem1,
             n0, n1, n2, n3, e0s, e1s, e2s, e3s, osem, wsem0, wsem1):
        c = lax.axis_index("c")
        s = lax.axis_index("s")
        dix = (dixa, dixb)
        rix = (rixa, rixb)
        buf = (bufa, bufs)
        isem = (isem0, isem1)
        rsem = (rsem0, rsem1)
        hsem = (hsem0, hsem1)
        nsem = (n0, n1, n2, n3)
        esem = (e0s, e1s, e2s, e3s)
        wsem = (wsem0, wsem1)
        cs = pl.ds(c * H, H)

        def gid_of(i):
            return s + i * NS

        _zero_acc(s, acc, bufa, n_nodes)
        plsc.subcore_barrier()

        def start_arr(i, b):
            g = gid_of(i)
            pltpu.async_copy(di_hbm.at[pl.ds(CH * g, CH)], dix[b], isem[b])
            pltpu.async_copy(eh_hbm.at[pl.ds(GK * g, GK), cs],
                             buf[b], hsem[b])

        def wait_wout(i, b):
            g = gid_of(i)
            pltpu.make_async_copy(
                buf[b], h_hbm.at[c, pl.ds(GK * g, GK)], wsem[b]).wait()

        def scat_group(i, b, issue_next, wait_w):
            g = gid_of(i)
            pltpu.make_async_copy(di_hbm.at[pl.ds(CH * g, CH)],
                                  dix[b], isem[b]).wait()
            pltpu.make_async_copy(eh_hbm.at[pl.ds(GK * g, GK), cs],
                                  buf[b], hsem[b]).wait()
            if issue_next:
                if wait_w:
                    wait_wout(i - 1, 1 - b)
                start_arr(i + 1, 1 - b)
            for j in range(CH):
                def op(r, sl):
                    buf[b][r, sl] = jnp.maximum(buf[b][r, sl], 0.0)
                _chunk_op(op, j)
                pltpu.sync_copy(buf[b].at[pl.ds(j * K, K)],
                                acc.at[dix[b].at[j]], add=True)
            pltpu.async_copy(buf[b], h_hbm.at[c, pl.ds(GK * g, GK)],
                             wsem[b])

        start_arr(0, 0)
        scat_group(0, 0, True, False)

        def sgrp(g, _):
            scat_group(2 * g + 1, 1, True, True)
            scat_group(2 * g + 2, 0, True, True)
            return _
        lax.fori_loop(0, (nmain - 3) // 2, sgrp, None)
        scat_group(nmain - 2, 1, True, True)
        scat_group(nmain - 1, 0, False, False)
        wait_wout(nmain - 2, 1)
        wait_wout(nmain - 1, 0)

        if nleft:
            @pl.when(s < nleft)
            def _():
                g = nmain * NS + s
                pltpu.sync_copy(di_hbm.at[pl.ds(CH * g, CH)], dixa)
                pltpu.sync_copy(eh_hbm.at[pl.ds(GK * g, GK), cs], bufa)
                for j in range(CH):
                    def op(r, sl):
                        bufa[r, sl] = jnp.maximum(bufa[r, sl], 0.0)
                    _chunk_op(op, j)
                    pltpu.sync_copy(bufa.at[pl.ds(j * K, K)],
                                    acc.at[dixa.at[j]], add=True)
                pltpu.sync_copy(bufa, h_hbm.at[c, pl.ds(GK * g, GK)])

        plsc.subcore_barrier()

        def start_idx(i, b):
            g = gid_of(i)
            pltpu.async_copy(si_hbm.at[pl.ds(CH * g, CH)], dix[b], isem[b])
            pltpu.async_copy(ri_hbm.at[pl.ds(CH * g, CH)], rix[b], rsem[b])

        def wait_em_out(i):
            g = gid_of(i)
            pltpu.make_async_copy(
                bufa, em_hbm.at[pl.ds(GK * g, GK), cs], osem).wait()

        def gath_body(b):
            cpn = [pltpu.async_copy(acc.at[dix[b].at[j]],
                                    bufa.at[pl.ds(j * K, K)], nsem[j])
                   for j in range(CH)]
            cpe = [pltpu.async_copy(h_hbm.at[c].at[rix[b].at[j]],
                                    bufs.at[pl.ds(j * K, K)], esem[j])
                   for j in range(CH)]
            for j in range(CH):
                cpn[j].wait()
                cpe[j].wait()

                def op(r, sl):
                    bufa[r, sl] = bufa[r, sl] - bufs[r, sl]
                _chunk_op(op, j)

        def gath_group(i, b, issue_idx, drain_out):
            g = gid_of(i)
            if drain_out:
                wait_em_out(i - 1)
            pltpu.make_async_copy(si_hbm.at[pl.ds(CH * g, CH)],
                                  dix[b], isem[b]).wait()
            pltpu.make_async_copy(ri_hbm.at[pl.ds(CH * g, CH)],
                                  rix[b], rsem[b]).wait()
            if issue_idx:
                start_idx(i + 1, 1 - b)
            gath_body(b)
            pltpu.async_copy(bufa, em_hbm.at[pl.ds(GK * g, GK), cs], osem)

        start_idx(0, 0)
        gath_group(0, 0, True, False)

        def ggrp(g, _):
            gath_group(2 * g + 1, 1, True, True)
            gath_group(2 * g + 2, 0, True, True)
            return _
        lax.fori_loop(0, (nmain - 3) // 2, ggrp, None)
        gath_group(nmain - 2, 1, True, True)
        gath_group(nmain - 1, 0, False, True)
        wait_em_out(nmain - 1)

        if nleft:
            @pl.when(s < nleft)
            def _():
                g = nmain * NS + s
                pltpu.sync_copy(si_hbm.at[pl.ds(CH * g, CH)], dixa)
                pltpu.sync_copy(ri_hbm.at[pl.ds(CH * g, CH)], rixa)
                gath_body(0)
                pltpu.sync_copy(bufa, em_hbm.at[pl.ds(GK * g, GK), cs])

    return body(eh, dest2d, src2d, rev2d)[0]


def _sc_final(eh, dest2d, n_nodes):
    e, d = eh.shape
    ngrp = dest2d.shape[0] // CH
    nmain = ngrp // NS
    nleft = ngrp - nmain * NS
    assert nmain % 2 == 1 and nmain >= 5

    @functools.partial(
        pl.kernel,
        out_type=_f32(n_nodes, d),
        mesh=_mesh(),
        compiler_params=_SC_PARAMS,
        scratch_types=[
            pltpu.VMEM_SHARED((n_nodes, H), jnp.float32),
            pltpu.VMEM((CH, K), jnp.int32),
            pltpu.VMEM((CH, K), jnp.int32),
            pltpu.VMEM((GK, H), jnp.float32),
            pltpu.VMEM((GK, H), jnp.float32),
            pltpu.SemaphoreType.DMA,
            pltpu.SemaphoreType.DMA,
            pltpu.SemaphoreType.DMA,
            pltpu.SemaphoreType.DMA,
        ],
    )
    def body(eh_hbm, di_hbm, nh_hbm,
             acc, dixa, dixb, bufa, bufb, isem0, isem1, hsem0, hsem1):
        c = lax.axis_index("c")
        s = lax.axis_index("s")
        dix = (dixa, dixb)
        buf = (bufa, bufb)
        isem = (isem0, isem1)
        hsem = (hsem0, hsem1)
        _zero_acc(s, acc, bufa, n_nodes)
        plsc.subcore_barrier()

        def gid_of(i):
            return s + i * NS

        def start_arr(i, b):
            g = gid_of(i)
            pltpu.async_copy(di_hbm.at[pl.ds(CH * g, CH)], dix[b], isem[b])
            pltpu.async_copy(eh_hbm.at[pl.ds(GK * g, GK), pl.ds(c * H, H)],
                             buf[b], hsem[b])

        def scat_group(i, b, issue_next):
            if issue_next:
                start_arr(i + 1, 1 - b)
            g = gid_of(i)
            pltpu.make_async_copy(di_hbm.at[pl.ds(CH * g, CH)],
                                  dix[b], isem[b]).wait()
            pltpu.make_async_copy(
                eh_hbm.at[pl.ds(GK * g, GK), pl.ds(c * H, H)],
                buf[b], hsem[b]).wait()
            for j in range(CH):
                pltpu.sync_copy(buf[b].at[pl.ds(j * K, K)],
                                acc.at[dix[b].at[j]], add=True)

        start_arr(0, 0)
        scat_group(0, 0, True)

        def sgrp(g, _):
            scat_group(2 * g + 1, 1, True)
            scat_group(2 * g + 2, 0, True)
            return _
        lax.fori_loop(0, (nmain - 3) // 2, sgrp, None)
        scat_group(nmain - 2, 1, True)
        scat_group(nmain - 1, 0, False)

        if nleft:
            @pl.when(s < nleft)
            def _():
                g = nmain * NS + s
                pltpu.sync_copy(di_hbm.at[pl.ds(CH * g, CH)], dixa)
                pltpu.sync_copy(
                    eh_hbm.at[pl.ds(GK * g, GK), pl.ds(c * H, H)], bufa)
                for j in range(CH):
                    pltpu.sync_copy(bufa.at[pl.ds(j * K, K)],
                                    acc.at[dixa.at[j]], add=True)

        plsc.subcore_barrier()
        per = n_nodes // NS
        r0 = s * per
        pltpu.sync_copy(acc.at[pl.ds(r0, per)],
                        nh_hbm.at[pl.ds(r0, per), pl.ds(c * H, H)])

    return body(eh, dest2d)


def _tc_layer(eh, em, W, b):
    e, d = eh.shape
    be = 3200
    grid = (e // be,)
    b2 = b.reshape(1, d)

    def mm_body(eh_ref, em_ref, w_ref, b_ref, out_ref):
        upd = jnp.dot(em_ref[...], w_ref[...],
                      preferred_element_type=jnp.float32)
        out_ref[...] = eh_ref[...] + upd + b_ref[...]

    return pl.pallas_call(
        mm_body,
        grid=grid,
        in_specs=[
            pl.BlockSpec((be, d), lambda i: (i, 0)),
            pl.BlockSpec((be, d), lambda i: (i, 0)),
            pl.BlockSpec((d, d), lambda i: (0, 0)),
            pl.BlockSpec((1, d), lambda i: (0, 0)),
        ],
        out_specs=pl.BlockSpec((be, d), lambda i: (i, 0)),
        out_shape=_f32(e, d),
    )(eh, em, W, b2)


def kernel(node_feats, edge_feats, edge_index, rev_index,
           W0, b0, W1, b1, W2, b2):
    n_nodes = node_feats.shape[0]
    e = edge_feats.shape[0]
    src2d = edge_index[0].reshape(e // K, K)
    dest2d = edge_index[1].reshape(e // K, K)
    rev2d = rev_index.reshape(e // K, K)
    eh = _sc_init(node_feats, edge_feats, src2d)
    for W, b in ((W0, b0), (W1, b1), (W2, b2)):
        em = _sc_message(eh, dest2d, src2d, rev2d, n_nodes)
        eh = _tc_layer(eh, em, W, b)
    node_hiddens = _sc_final(eh, dest2d, n_nodes)
    return (node_hiddens, eh)


# speedup vs baseline: 3.4361x; 1.0271x over previous
import functools

import jax
import jax.numpy as jnp
from jax import lax
from jax.experimental import pallas as pl
from jax.experimental.pallas import tpu as pltpu
from jax.experimental.pallas import tpu_sc as plsc

NC = 2
NS = 16
LN = 16
K = 128
CH = 4
GK = K * CH
H = 64
D = 2 * H


def _mesh():
    return plsc.VectorSubcoreMesh(core_axis_name="c", subcore_axis_name="s")


_SC_PARAMS = pltpu.CompilerParams(use_tc_tiling_on_sc=False)


def _f32(*shape):
    return jax.ShapeDtypeStruct(shape, jnp.float32)


def _chunk_op(body, j):
    def outer(r, _):
        for q in range(H // LN):
            body(j * K + r, pl.ds(q * LN, LN))
        return _
    lax.fori_loop(0, K, outer, None)


def _zero_acc(s, acc, zbuf, n_rows):
    def z(r, _):
        for q in range(H // LN):
            zbuf[r, pl.ds(q * LN, LN)] = jnp.zeros((LN,), jnp.float32)
        return _
    lax.fori_loop(0, K, z, None)
    per = n_rows // NS
    r0 = s * per
    off = 0
    while off < per:
        kk = min(K, per - off)
        pltpu.sync_copy(zbuf.at[pl.ds(0, kk)], acc.at[pl.ds(r0 + off, kk)])
        off += kk


def _sc_init(node_feats, edge_feats, src2d):
    n, d = node_feats.shape
    ngrp = src2d.shape[0] // CH
    e = src2d.shape[0] * K
    nmain = ngrp // NS
    nleft = ngrp - nmain * NS
    assert nmain % 2 == 1 and nmain >= 5 and n % NS == 0

    @functools.partial(
        pl.kernel,
        out_type=_f32(e, D),
        mesh=_mesh(),
        compiler_params=_SC_PARAMS,
        scratch_types=[
            pltpu.VMEM_SHARED((n, H), jnp.float32),
            pltpu.VMEM((CH, K), jnp.int32),
            pltpu.VMEM((CH, K), jnp.int32),
            pltpu.VMEM((K, H), jnp.float32),
            pltpu.VMEM((K, H), jnp.float32),
            pltpu.VMEM((GK, H), jnp.float32),
            pltpu.VMEM((GK, H), jnp.float32),
            pltpu.SemaphoreType.DMA,
            pltpu.SemaphoreType.DMA,
            pltpu.SemaphoreType.DMA,
            pltpu.SemaphoreType.DMA,
            pltpu.SemaphoreType.DMA,
            pltpu.SemaphoreType.DMA,
            pltpu.SemaphoreType.DMA,
            pltpu.SemaphoreType.DMA,
        ],
    )
    def body(nf_hbm, ef_hbm, si_hbm, eh_hbm,
             nf_sh, sidxa, sidxb, nfp0, nfp1, efa, efb,
             isem0, isem1, esem0, esem1,
             n0, n1, osem0, osem1):
        c = lax.axis_index("c")
        s = lax.axis_index("s")
        sidx = (sidxa, sidxb)
        nfp = (nfp0, nfp1)
        efx = (efa, efb)
        isem = (isem0, isem1)
        esem = (esem0, esem1)
        nsem = (n0, n1)
        osem = (osem0, osem1)
        cs = pl.ds(c * H, H)

        per = n // NS
        r0 = s * per
        pltpu.sync_copy(nf_hbm.at[pl.ds(r0, per), cs],
                        nf_sh.at[pl.ds(r0, per)])
        plsc.subcore_barrier()

        def gid_of(i):
            return s + i * NS

        def start_arr(i, b):
            g = gid_of(i)
            pltpu.async_copy(si_hbm.at[pl.ds(CH * g, CH)], sidx[b], isem[b])
            pltpu.async_copy(ef_hbm.at[pl.ds(GK * g, GK), cs],
                             efx[b], esem[b])

        def wait_arr(i, b):
            g = gid_of(i)
            pltpu.make_async_copy(si_hbm.at[pl.ds(CH * g, CH)],
                                  sidx[b], isem[b]).wait()
            pltpu.make_async_copy(ef_hbm.at[pl.ds(GK * g, GK), cs],
                                  efx[b], esem[b]).wait()

        def wait_out(i, b):
            g = gid_of(i)
            pltpu.make_async_copy(
                efx[b], eh_hbm.at[pl.ds(GK * g, GK), cs], osem[b]).wait()

        def gath_adds(b):
            cps = [None] * CH
            cps[0] = pltpu.async_copy(nf_sh.at[sidx[b].at[0]], nfp0, nsem[0])
            for j in range(CH):
                if j + 1 < CH:
                    cps[j + 1] = pltpu.async_copy(
                        nf_sh.at[sidx[b].at[j + 1]],
                        nfp[(j + 1) % 2], nsem[(j + 1) % 2])
                cps[j].wait()
                nfj = nfp[j % 2]

                def op(r, sl):
                    efx[b][j * K + r, sl] = efx[b][j * K + r, sl] + nfj[r, sl]

                def outer(r, _):
                    for q in range(H // LN):
                        op(r, pl.ds(q * LN, LN))
                    return _
                lax.fori_loop(0, K, outer, None)

        def group(i, b, issue_next, wait_o):
            wait_arr(i, b)
            if issue_next:
                if wait_o:
                    wait_out(i - 1, 1 - b)
                start_arr(i + 1, 1 - b)
            gath_adds(b)
            g = gid_of(i)
            pltpu.async_copy(efx[b], eh_hbm.at[pl.ds(GK * g, GK), cs],
                             osem[b])

        start_arr(0, 0)
        group(0, 0, True, False)

        def grp(g, _):
            group(2 * g + 1, 1, True, True)
            group(2 * g + 2, 0, True, True)
            return _
        lax.fori_loop(0, (nmain - 3) // 2, grp, None)
        group(nmain - 2, 1, True, True)
        group(nmain - 1, 0, False, False)
        wait_out(nmain - 2, 1)
        wait_out(nmain - 1, 0)

        if nleft:
            @pl.when(s < nleft)
            def _():
                g = nmain * NS + s
                pltpu.sync_copy(si_hbm.at[pl.ds(CH * g, CH)], sidxa)
                pltpu.sync_copy(ef_hbm.at[pl.ds(GK * g, GK), cs], efa)
                gath_adds(0)
                pltpu.sync_copy(efa, eh_hbm.at[pl.ds(GK * g, GK), cs])

    return body(node_feats, edge_feats, src2d)


def _sc_message(eh, dest2d, src2d, rev2d, n_nodes, pre_relu):
    ngrp = dest2d.shape[0] // CH
    e = dest2d.shape[0] * K
    nmain = ngrp // NS
    nleft = ngrp - nmain * NS
    assert nmain % 2 == 1 and nmain >= 5

    @functools.partial(
        pl.kernel,
        out_type=(_f32(e, D), _f32(NC, e, H)),
        mesh=_mesh(),
        compiler_params=_SC_PARAMS,
        scratch_types=[
            pltpu.VMEM_SHARED((n_nodes, H), jnp.float32),
            pltpu.VMEM((CH, K), jnp.int32),
            pltpu.VMEM((CH, K), jnp.int32),
            pltpu.VMEM((CH, K), jnp.int32),
            pltpu.VMEM((CH, K), jnp.int32),
            pltpu.VMEM((GK, H), jnp.float32),
            pltpu.VMEM((GK, H), jnp.float32),
            pltpu.SemaphoreType.DMA,
            pltpu.SemaphoreType.DMA,
            pltpu.SemaphoreType.DMA,
            pltpu.SemaphoreType.DMA,
            pltpu.SemaphoreType.DMA,
            pltpu.SemaphoreType.DMA,
            pltpu.SemaphoreType.DMA,
            pltpu.SemaphoreType.DMA,
            pltpu.SemaphoreType.DMA,
            pltpu.SemaphoreType.DMA,
            pltpu.SemaphoreType.DMA,
            pltpu.SemaphoreType.DMA,
            pltpu.SemaphoreType.DMA,
            pltpu.SemaphoreType.DMA,
            pltpu.SemaphoreType.DMA,
            pltpu.SemaphoreType.DMA,
            pltpu.SemaphoreType.DMA,
        ],
    )
    def body(eh_hbm, di_hbm, si_hbm, ri_hbm, em_hbm, h_hbm,
             acc, dixa, dixb, rixa, rixb, bufa, bufs,
             isem0, isem1, rsem0, rsem1, hsem0, hsem1,
             n0, n1, n2, n3, e0s, e1s, e2s, e3s, osem, wsem0, wsem1):
        c = lax.axis_index("c")
        s = lax.axis_index("s")
        dix = (dixa, dixb)
        rix = (rixa, rixb)
        buf = (bufa, bufs)
        isem = (isem0, isem1)
        rsem = (rsem0, rsem1)
        hsem = (hsem0, hsem1)
        nsem = (n0, n1, n2, n3)
        esem = (e0s, e1s, e2s, e3s)
        wsem = (wsem0, wsem1)
        cs = pl.ds(c * H, H)

        def gid_of(i):
            return s + i * NS

        _zero_acc(s, acc, bufa, n_nodes)
        plsc.subcore_barrier()

        def start_arr(i, b):
            g = gid_of(i)
            pltpu.async_copy(di_hbm.at[pl.ds(CH * g, CH)], dix[b], isem[b])
            pltpu.async_copy(eh_hbm.at[pl.ds(GK * g, GK), cs],
                             buf[b], hsem[b])

        def wait_wout(i, b):
            g = gid_of(i)
            pltpu.make_async_copy(
                buf[b], h_hbm.at[c, pl.ds(GK * g, GK)], wsem[b]).wait()

        def scat_group(i, b, issue_next, wait_w):
            g = gid_of(i)
            pltpu.make_async_copy(di_hbm.at[pl.ds(CH * g, CH)],
                                  dix[b], isem[b]).wait()
            pltpu.make_async_copy(eh_hbm.at[pl.ds(GK * g, GK), cs],
                                  buf[b], hsem[b]).wait()
            if issue_next:
                if wait_w:
                    wait_wout(i - 1, 1 - b)
                start_arr(i + 1, 1 - b)
            for j in range(CH):
                if not pre_relu:
                    def op(r, sl):
                        buf[b][r, sl] = jnp.maximum(buf[b][r, sl], 0.0)
                    _chunk_op(op, j)
                pltpu.sync_copy(buf[b].at[pl.ds(j * K, K)],
                                acc.at[dix[b].at[j]], add=True)
            pltpu.async_copy(buf[b], h_hbm.at[c, pl.ds(GK * g, GK)],
                             wsem[b])

        start_arr(0, 0)
        scat_group(0, 0, True, False)

        def sgrp(g, _):
            scat_group(2 * g + 1, 1, True, True)
            scat_group(2 * g + 2, 0, True, True)
            return _
        lax.fori_loop(0, (nmain - 3) // 2, sgrp, None)
        scat_group(nmain - 2, 1, True, True)
        scat_group(nmain - 1, 0, False, False)
        wait_wout(nmain - 2, 1)
        wait_wout(nmain - 1, 0)

        if nleft:
            @pl.when(s < nleft)
            def _():
                g = nmain * NS + s
                pltpu.sync_copy(di_hbm.at[pl.ds(CH * g, CH)], dixa)
                pltpu.sync_copy(eh_hbm.at[pl.ds(GK * g, GK), cs], bufa)
                for j in range(CH):
                    if not pre_relu:
                        def op(r, sl):
                            bufa[r, sl] = jnp.maximum(bufa[r, sl], 0.0)
                        _chunk_op(op, j)
                    pltpu.sync_copy(bufa.at[pl.ds(j * K, K)],
                                    acc.at[dixa.at[j]], add=True)
                pltpu.sync_copy(bufa, h_hbm.at[c, pl.ds(GK * g, GK)])

        plsc.subcore_barrier()

        def start_idx(i, b):
            g = gid_of(i)
            pltpu.async_copy(si_hbm.at[pl.ds(CH * g, CH)], dix[b], isem[b])
            pltpu.async_copy(ri_hbm.at[pl.ds(CH * g, CH)], rix[b], rsem[b])

        def wait_em_out(i):
            g = gid_of(i)
            pltpu.make_async_copy(
                bufa, em_hbm.at[pl.ds(GK * g, GK), cs], osem).wait()

        def gath_body(b):
            cpn = [pltpu.async_copy(acc.at[dix[b].at[j]],
                                    bufa.at[pl.ds(j * K, K)], nsem[j])
                   for j in range(CH)]
            cpe = [pltpu.async_copy(h_hbm.at[c].at[rix[b].at[j]],
                                    bufs.at[pl.ds(j * K, K)], esem[j])
                   for j in range(CH)]
            for j in range(CH):
                cpn[j].wait()
                cpe[j].wait()

                def op(r, sl):
                    bufa[r, sl] = bufa[r, sl] - bufs[r, sl]
                _chunk_op(op, j)

        def gath_group(i, b, issue_idx, drain_out):
            g = gid_of(i)
            if drain_out:
                wait_em_out(i - 1)
            pltpu.make_async_copy(si_hbm.at[pl.ds(CH * g, CH)],
                                  dix[b], isem[b]).wait()
            pltpu.make_async_copy(ri_hbm.at[pl.ds(CH * g, CH)],
                                  rix[b], rsem[b]).wait()
            if issue_idx:
                start_idx(i + 1, 1 - b)
            gath_body(b)
            pltpu.async_copy(bufa, em_hbm.at[pl.ds(GK * g, GK), cs], osem)

        start_idx(0, 0)
        gath_group(0, 0, True, False)

        def ggrp(g, _):
            gath_group(2 * g + 1, 1, True, True)
            gath_group(2 * g + 2, 0, True, True)
            return _
        lax.fori_loop(0, (nmain - 3) // 2, ggrp, None)
        gath_group(nmain - 2, 1, True, True)
        gath_group(nmain - 1, 0, False, True)
        wait_em_out(nmain - 1)

        if nleft:
            @pl.when(s < nleft)
            def _():
                g = nmain * NS + s
                pltpu.sync_copy(si_hbm.at[pl.ds(CH * g, CH)], dixa)
                pltpu.sync_copy(ri_hbm.at[pl.ds(CH * g, CH)], rixa)
                gath_body(0)
                pltpu.sync_copy(bufa, em_hbm.at[pl.ds(GK * g, GK), cs])

    return body(eh, dest2d, src2d, rev2d)[0]


def _sc_final(eh, dest2d, n_nodes):
    e, d = eh.shape
    ngrp = dest2d.shape[0] // CH
    nmain = ngrp // NS
    nleft = ngrp - nmain * NS
    assert nmain % 2 == 1 and nmain >= 5

    @functools.partial(
        pl.kernel,
        out_type=_f32(n_nodes, d),
        mesh=_mesh(),
        compiler_params=_SC_PARAMS,
        scratch_types=[
            pltpu.VMEM_SHARED((n_nodes, H), jnp.float32),
            pltpu.VMEM((CH, K), jnp.int32),
            pltpu.VMEM((CH, K), jnp.int32),
            pltpu.VMEM((GK, H), jnp.float32),
            pltpu.VMEM((GK, H), jnp.float32),
            pltpu.SemaphoreType.DMA,
            pltpu.SemaphoreType.DMA,
            pltpu.SemaphoreType.DMA,
            pltpu.SemaphoreType.DMA,
        ],
    )
    def body(eh_hbm, di_hbm, nh_hbm,
             acc, dixa, dixb, bufa, bufb, isem0, isem1, hsem0, hsem1):
        c = lax.axis_index("c")
        s = lax.axis_index("s")
        dix = (dixa, dixb)
        buf = (bufa, bufb)
        isem = (isem0, isem1)
        hsem = (hsem0, hsem1)
        _zero_acc(s, acc, bufa, n_nodes)
        plsc.subcore_barrier()

        def gid_of(i):
            return s + i * NS

        def start_arr(i, b):
            g = gid_of(i)
            pltpu.async_copy(di_hbm.at[pl.ds(CH * g, CH)], dix[b], isem[b])
            pltpu.async_copy(eh_hbm.at[pl.ds(GK * g, GK), pl.ds(c * H, H)],
                             buf[b], hsem[b])

        def scat_group(i, b, issue_next):
            if issue_next:
                start_arr(i + 1, 1 - b)
            g = gid_of(i)
            pltpu.make_async_copy(di_hbm.at[pl.ds(CH * g, CH)],
                                  dix[b], isem[b]).wait()
            pltpu.make_async_copy(
                eh_hbm.at[pl.ds(GK * g, GK), pl.ds(c * H, H)],
                buf[b], hsem[b]).wait()
            for j in range(CH):
                pltpu.sync_copy(buf[b].at[pl.ds(j * K, K)],
                                acc.at[dix[b].at[j]], add=True)

        start_arr(0, 0)
        scat_group(0, 0, True)

        def sgrp(g, _):
            scat_group(2 * g + 1, 1, True)
            scat_group(2 * g + 2, 0, True)
            return _
        lax.fori_loop(0, (nmain - 3) // 2, sgrp, None)
        scat_group(nmain - 2, 1, True)
        scat_group(nmain - 1, 0, False)

        if nleft:
            @pl.when(s < nleft)
            def _():
                g = nmain * NS + s
                pltpu.sync_copy(di_hbm.at[pl.ds(CH * g, CH)], dixa)
                pltpu.sync_copy(
                    eh_hbm.at[pl.ds(GK * g, GK), pl.ds(c * H, H)], bufa)
                for j in range(CH):
                    pltpu.sync_copy(bufa.at[pl.ds(j * K, K)],
                                    acc.at[dixa.at[j]], add=True)

        plsc.subcore_barrier()
        per = n_nodes // NS
        r0 = s * per
        pltpu.sync_copy(acc.at[pl.ds(r0, per)],
                        nh_hbm.at[pl.ds(r0, per), pl.ds(c * H, H)])

    return body(eh, dest2d)


def _tc_layer(eh, em, W, b, emit_relu):
    e, d = eh.shape
    be = 3200
    grid = (e // be,)
    b2 = b.reshape(1, d)

    if emit_relu:
        def mm_body(eh_ref, em_ref, w_ref, b_ref, out_ref, hr_ref):
            upd = jnp.dot(em_ref[...], w_ref[...],
                          preferred_element_type=jnp.float32)
            new = eh_ref[...] + upd + b_ref[...]
            out_ref[...] = new
            hr_ref[...] = jnp.maximum(new, 0.0)
        out_specs = (pl.BlockSpec((be, d), lambda i: (i, 0)),
                     pl.BlockSpec((be, d), lambda i: (i, 0)))
        out_shape = (_f32(e, d), _f32(e, d))
    else:
        def mm_body(eh_ref, em_ref, w_ref, b_ref, out_ref):
            upd = jnp.dot(em_ref[...], w_ref[...],
                          preferred_element_type=jnp.float32)
            out_ref[...] = eh_ref[...] + upd + b_ref[...]
        out_specs = pl.BlockSpec((be, d), lambda i: (i, 0))
        out_shape = _f32(e, d)

    return pl.pallas_call(
        mm_body,
        grid=grid,
        in_specs=[
            pl.BlockSpec((be, d), lambda i: (i, 0)),
            pl.BlockSpec((be, d), lambda i: (i, 0)),
            pl.BlockSpec((d, d), lambda i: (0, 0)),
            pl.BlockSpec((1, d), lambda i: (0, 0)),
        ],
        out_specs=out_specs,
        out_shape=out_shape,
    )(eh, em, W, b2)


def kernel(node_feats, edge_feats, edge_index, rev_index,
           W0, b0, W1, b1, W2, b2):
    n_nodes = node_feats.shape[0]
    e = edge_feats.shape[0]
    src2d = edge_index[0].reshape(e // K, K)
    dest2d = edge_index[1].reshape(e // K, K)
    rev2d = rev_index.reshape(e // K, K)
    eh = _sc_init(node_feats, edge_feats, src2d)
    hin, pre = eh, False
    for i, (W, b) in enumerate(((W0, b0), (W1, b1), (W2, b2))):
        em = _sc_message(hin, dest2d, src2d, rev2d, n_nodes, pre_relu=pre)
        if i < 2:
            eh, hin = _tc_layer(eh, em, W, b, emit_relu=True)
            pre = True
        else:
            eh = _tc_layer(eh, em, W, b, emit_relu=False)
    node_hiddens = _sc_final(eh, dest2d, n_nodes)
    return (node_hiddens, eh)
